# Initial kernel scaffold; baseline (speedup 1.0000x reference)
import jax
import jax.numpy as jnp
from jax.experimental import pallas as pl


def kernel(patch_embs, edge_index, edge_attr, Wl1, bl1, Wr1, br1, att1, bias1, g1, be1, Wl2, bl2, Wr2, br2, att2, bias2, g2, be2, Wl3, bl3, Wr3, br3, att3, bias3, g3, be3):
    raise NotImplementedError("write your pallas kernel here")


# speedup vs baseline: 20.5881x; 20.5881x over previous
import functools

import jax
import jax.numpy as jnp
from jax import lax
from jax.experimental import pallas as pl
from jax.experimental.pallas import tpu as pltpu
from jax.experimental.pallas import tpu_sc as plsc

N = 50000
E = 800000
D = 32
H = 8
C = 32
HC = H * C

NC, NS, L = 2, 16, 16
NW = NC * NS
EPW = 25600
E_PAD = NW * EPW
BE = 64
NCHUNK = EPW // BE
N_PAD = 50176
RPT = N_PAD // NS

_mesh = plsc.VectorSubcoreMesh(core_axis_name="c", subcore_axis_name="s")


def _t1_body(x_ref, wl_ref, bl_ref, wr_ref, br_ref, att_ref,
             xl_ref, xr_ref, al_ref, lmax_ref):
    i = pl.program_id(0)
    x = x_ref[...]
    xl = jnp.dot(x, wl_ref[...], preferred_element_type=jnp.float32) + bl_ref[...]
    xr = jnp.dot(x, wr_ref[...], preferred_element_type=jnp.float32) + br_ref[...]
    xl_ref[...] = xl
    xr_ref[...] = xr
    r = x.shape[0]
    s = (xl + xr).reshape(r, H, C)
    m = jnp.where(s >= 0, s, 0.2 * s)
    al = jnp.sum(m * att_ref[...][None], axis=-1)
    al_ref[...] = al
    bm = jnp.max(al, axis=0, keepdims=True)
    prev = jnp.where(i == 0, jnp.full((1, H), -3.0e38, jnp.float32), lmax_ref[...])
    lmax_ref[...] = jnp.maximum(prev, bm)


def _t1(x, wl, bl, wr, br, att):
    r = 2000
    grid = N // r
    return pl.pallas_call(
        _t1_body,
        grid=(grid,),
        in_specs=[
            pl.BlockSpec((r, D), lambda i: (i, 0)),
            pl.BlockSpec((D, HC), lambda i: (0, 0)),
            pl.BlockSpec((1, HC), lambda i: (0, 0)),
            pl.BlockSpec((D, HC), lambda i: (0, 0)),
            pl.BlockSpec((1, HC), lambda i: (0, 0)),
            pl.BlockSpec((H, C), lambda i: (0, 0)),
        ],
        out_specs=[
            pl.BlockSpec((r, HC), lambda i: (i, 0)),
            pl.BlockSpec((r, HC), lambda i: (i, 0)),
            pl.BlockSpec((r, H), lambda i: (i, 0)),
            pl.BlockSpec((1, H), lambda i: (0, 0)),
        ],
        out_shape=[
            jax.ShapeDtypeStruct((N, HC), jnp.float32),
            jax.ShapeDtypeStruct((N, HC), jnp.float32),
            jax.ShapeDtypeStruct((N, H), jnp.float32),
            jax.ShapeDtypeStruct((1, H), jnp.float32),
        ],
    )(x, wl, bl.reshape(1, HC), wr, br.reshape(1, HC), att)


def _t2_body(a0_ref, a1_ref, al_ref, gmax_ref, xl_ref, den_ref, lc_ref):
    asum = a0_ref[...] + a1_ref[...]
    sae = jnp.exp(al_ref[...] - gmax_ref[...])
    den = 1.0 / (asum + sae)
    den_ref[...] = jnp.concatenate([den, jnp.zeros_like(den)], axis=1)
    w = sae * den
    r = w.shape[0]
    xl3 = xl_ref[...].reshape(r, H, C)
    lc_ref[...] = jnp.sum(w[:, :, None] * xl3, axis=1)


def _t2(a0, a1, aloop, gmax, xl):
    r = 400
    grid = N // r
    return pl.pallas_call(
        _t2_body,
        grid=(grid,),
        in_specs=[
            pl.BlockSpec((r, H), lambda i: (i, 0)),
            pl.BlockSpec((r, H), lambda i: (i, 0)),
            pl.BlockSpec((r, H), lambda i: (i, 0)),
            pl.BlockSpec((1, H), lambda i: (0, 0)),
            pl.BlockSpec((r, HC), lambda i: (i, 0)),
        ],
        out_specs=[
            pl.BlockSpec((r, 2 * H), lambda i: (i, 0)),
            pl.BlockSpec((r, C), lambda i: (i, 0)),
        ],
        out_shape=[
            jax.ShapeDtypeStruct((N, 2 * H), jnp.float32),
            jax.ShapeDtypeStruct((N, C), jnp.float32),
        ],
    )(a0, a1, aloop, gmax, xl)


def _t3a_body(p0_ref, p1_ref, lc_ref, bias_ref, xp_ref, y_ref, st_ref, *, act):
    i = pl.program_id(0)
    tot = (p0_ref[...] + p1_ref[...] + lc_ref[...]) * (1.0 / H) + bias_ref[...]
    if act:
        tot = jnp.where(tot >= 0, tot, 0.01 * tot)
    y = tot + xp_ref[...]
    y_ref[...] = y
    s = jnp.concatenate([jnp.sum(y, axis=0, keepdims=True),
                         jnp.sum(y * y, axis=0, keepdims=True)], axis=0)
    prev = jnp.where(i == 0, jnp.zeros((2, C), jnp.float32), st_ref[...])
    st_ref[...] = prev + s


def _t3a(p0, p1, lc, bias, xp, act):
    r = 2000
    grid = N // r
    return pl.pallas_call(
        functools.partial(_t3a_body, act=act),
        grid=(grid,),
        in_specs=[
            pl.BlockSpec((r, C), lambda i: (i, 0)),
            pl.BlockSpec((r, C), lambda i: (i, 0)),
            pl.BlockSpec((r, C), lambda i: (i, 0)),
            pl.BlockSpec((1, C), lambda i: (0, 0)),
            pl.BlockSpec((r, C), lambda i: (i, 0)),
        ],
        out_specs=[
            pl.BlockSpec((r, C), lambda i: (i, 0)),
            pl.BlockSpec((2, C), lambda i: (0, 0)),
        ],
        out_shape=[
            jax.ShapeDtypeStruct((N, C), jnp.float32),
            jax.ShapeDtypeStruct((2, C), jnp.float32),
        ],
    )(p0, p1, lc, bias.reshape(1, C), xp)


def _t3b_body(y_ref, st_ref, g_ref, be_ref, o_ref):
    st = st_ref[...]
    mu = st[0:1] / N
    var = st[1:2] / N - mu * mu
    scale = g_ref[...] / jnp.sqrt(var + 1e-5)
    o_ref[...] = (y_ref[...] - mu) * scale + be_ref[...]


def _t3b(y, st, g, be):
    r = 2000
    grid = N // r
    return pl.pallas_call(
        _t3b_body,
        grid=(grid,),
        in_specs=[
            pl.BlockSpec((r, C), lambda i: (i, 0)),
            pl.BlockSpec((2, C), lambda i: (0, 0)),
            pl.BlockSpec((1, C), lambda i: (0, 0)),
            pl.BlockSpec((1, C), lambda i: (0, 0)),
        ],
        out_specs=pl.BlockSpec((r, C), lambda i: (i, 0)),
        out_shape=jax.ShapeDtypeStruct((N, C), jnp.float32),
    )(y, st, g.reshape(1, C), be.reshape(1, C))


def _e1_body(xl_hbm, xr_hbm, src_hbm, dst_hbm, att_hbm,
             alpha_hbm, wmax_hbm,
             srcv, dstv, xlv, xrv, abuf, attv, wbuf, sem1, sem2):
    cid = lax.axis_index("c")
    sid = lax.axis_index("s")
    wid = sid * NC + cid
    base = wid * EPW
    pltpu.sync_copy(att_hbm, attv)
    lanes = lax.iota(jnp.int32, L)
    neg = jnp.full((L,), -3.0e38, jnp.float32)

    def chunk(gc, carry):
        off = base + gc * BE
        pltpu.sync_copy(src_hbm.at[pl.ds(off, BE)], srcv)
        pltpu.sync_copy(dst_hbm.at[pl.ds(off, BE)], dstv)
        cp1 = pltpu.async_copy(xl_hbm.at[srcv], xlv, sem1)
        cp2 = pltpu.async_copy(xr_hbm.at[dstv], xrv, sem2)
        cp1.wait()
        cp2.wait()

        def group(gg, mx):
            rows = gg * L + lanes
            newmx = []
            for h in range(H):
                va = attv[h, pl.ds(0, L)]
                vb = attv[h, pl.ds(L, L)]
                acc = jnp.zeros((L,), jnp.float32)
                for cc in range(C):
                    col = jnp.full((L,), h * C + cc, jnp.int32)
                    a = plsc.load_gather(xlv, [rows, col])
                    b = plsc.load_gather(xrv, [rows, col])
                    s = a + b
                    m = jnp.maximum(s, 0.2 * s)
                    att_s = va[cc] if cc < L else vb[cc - L]
                    acc = acc + m * att_s
                plsc.store_scatter(abuf, [rows * H + h], acc)
                newmx.append(jnp.maximum(mx[h], acc))
            return tuple(newmx)

        carry = lax.fori_loop(0, BE // L, group, carry)
        pltpu.sync_copy(abuf, alpha_hbm.at[pl.ds(off * H, BE * H)])
        return carry

    carry = lax.fori_loop(0, NCHUNK, chunk, tuple(neg for _ in range(H)))
    for h in range(H):
        wbuf[h, :] = carry[h]
    pltpu.sync_copy(wbuf, wmax_hbm.at[wid])


@functools.partial(
    pl.kernel,
    out_type=[
        jax.ShapeDtypeStruct((E_PAD * H,), jnp.float32),
        jax.ShapeDtypeStruct((NW, H, L), jnp.float32),
    ],
    mesh=_mesh,
    compiler_params=pltpu.CompilerParams(use_tc_tiling_on_sc=False, needs_layout_passes=False),
    scratch_types=[
        pltpu.VMEM((BE,), jnp.int32),
        pltpu.VMEM((BE,), jnp.int32),
        pltpu.VMEM((BE, HC), jnp.float32),
        pltpu.VMEM((BE, HC), jnp.float32),
        pltpu.VMEM((BE * H,), jnp.float32),
        pltpu.VMEM((H, C), jnp.float32),
        pltpu.VMEM((H, L), jnp.float32),
        pltpu.SemaphoreType.DMA,
        pltpu.SemaphoreType.DMA,
    ],
)
def _e1(xl_hbm, xr_hbm, src_hbm, dst_hbm, att_hbm, alpha_hbm, wmax_hbm,
        srcv, dstv, xlv, xrv, abuf, attv, wbuf, sem1, sem2):
    _e1_body(xl_hbm, xr_hbm, src_hbm, dst_hbm, att_hbm, alpha_hbm, wmax_hbm,
             srcv, dstv, xlv, xrv, abuf, attv, wbuf, sem1, sem2)


def _e2_body(alpha_hbm, dst_hbm, gmax_hbm, z16_hbm, asum_hbm,
             dstv, albuf, upbuf, gmv, asum_sh, sem):
    cid = lax.axis_index("c")
    sid = lax.axis_index("s")
    wid = sid * NC + cid
    base = wid * EPW
    pltpu.sync_copy(z16_hbm.at[pl.ds(sid * RPT, RPT)],
                    asum_sh.at[pl.ds(sid * RPT, RPT)])
    plsc.subcore_barrier()
    pltpu.sync_copy(gmax_hbm, gmv)
    lanes = lax.iota(jnp.int32, L)
    rowpat = lanes >> 3
    colpat = lanes & 7
    zv = jnp.zeros((L,), jnp.float32)

    def zb(i, _):
        upbuf[i, :] = zv
        return 0

    lax.fori_loop(0, BE, zb, 0)
    gm = gmv[...]

    def chunk(gc, _):
        off = base + gc * BE
        pltpu.sync_copy(dst_hbm.at[pl.ds(off, BE)], dstv)
        pltpu.sync_copy(alpha_hbm.at[pl.ds(off * H, BE * H)], albuf)
        for j in range(BE * H // L):
            av = albuf[pl.ds(j * L, L)]
            ae = jnp.exp(av - gm)
            plsc.store_scatter(upbuf, [2 * j + rowpat, colpat], ae)
        pltpu.async_copy(upbuf, asum_sh.at[dstv], sem, add=True).wait()
        return 0

    lax.fori_loop(0, NCHUNK, chunk, 0)
    plsc.subcore_barrier()
    pltpu.sync_copy(asum_sh.at[pl.ds(sid * RPT, RPT)],
                    asum_hbm.at[cid, pl.ds(sid * RPT, RPT)])


@functools.partial(
    pl.kernel,
    out_type=jax.ShapeDtypeStruct((NC, N_PAD, 2 * H), jnp.float32),
    mesh=_mesh,
    compiler_params=pltpu.CompilerParams(use_tc_tiling_on_sc=False, needs_layout_passes=False),
    scratch_types=[
        pltpu.VMEM((BE,), jnp.int32),
        pltpu.VMEM((BE * H,), jnp.float32),
        pltpu.VMEM((BE, 2 * H), jnp.float32),
        pltpu.VMEM((L,), jnp.float32),
        pltpu.VMEM_SHARED((N_PAD, 2 * H), jnp.float32),
        pltpu.SemaphoreType.DMA,
    ],
)
def _e2(alpha_hbm, dst_hbm, gmax_hbm, z16_hbm, asum_hbm,
        dstv, albuf, upbuf, gmv, asum_sh, sem):
    _e2_body(alpha_hbm, dst_hbm, gmax_hbm, z16_hbm, asum_hbm,
             dstv, albuf, upbuf, gmv, asum_sh, sem)


def _e3_body(alpha_hbm, gmax_hbm, den_hbm, xl_hbm, src_hbm, dst_hbm, z32_hbm,
             out_hbm,
             srcv, dstv, albuf, denb, xlv, wbuf, ctb, gmv, out_sh,
             sem1, sem2, sem3):
    cid = lax.axis_index("c")
    sid = lax.axis_index("s")
    wid = sid * NC + cid
    base = wid * EPW
    pltpu.sync_copy(z32_hbm.at[pl.ds(sid * RPT, RPT)],
                    out_sh.at[pl.ds(sid * RPT, RPT)])
    plsc.subcore_barrier()
    pltpu.sync_copy(gmax_hbm, gmv)
    lanes = lax.iota(jnp.int32, L)
    rowpat = lanes >> 3
    colpat = lanes & 7
    gm = gmv[...]

    def chunk(gc, _):
        off = base + gc * BE
        pltpu.sync_copy(src_hbm.at[pl.ds(off, BE)], srcv)
        pltpu.sync_copy(dst_hbm.at[pl.ds(off, BE)], dstv)
        cp1 = pltpu.async_copy(xl_hbm.at[srcv], xlv, sem1)
        cp2 = pltpu.async_copy(den_hbm.at[dstv], denb, sem2)
        pltpu.sync_copy(alpha_hbm.at[pl.ds(off * H, BE * H)], albuf)
        cp2.wait()
        for j in range(BE * H // L):
            av = albuf[pl.ds(j * L, L)]
            ae = jnp.exp(av - gm)
            dv = plsc.load_gather(denb, [2 * j + rowpat, colpat])
            wbuf[pl.ds(j * L, L)] = ae * dv
        cp1.wait()

        def edge(ep, _):
            e0 = 2 * ep
            wv = wbuf[pl.ds(e0 * H, L)]
            for k in range(2):
                e = e0 + k
                c0 = jnp.zeros((L,), jnp.float32)
                c1 = jnp.zeros((L,), jnp.float32)
                for h in range(H):
                    ws = wv[k * H + h]
                    c0 = c0 + ws * xlv[e, pl.ds(h * C, L)]
                    c1 = c1 + ws * xlv[e, pl.ds(h * C + L, L)]
                ctb[e, pl.ds(0, L)] = c0
                ctb[e, pl.ds(L, L)] = c1
            return 0

        lax.fori_loop(0, BE // 2, edge, 0)
        pltpu.async_copy(ctb, out_sh.at[dstv], sem3, add=True).wait()
        return 0

    lax.fori_loop(0, NCHUNK, chunk, 0)
    plsc.subcore_barrier()
    pltpu.sync_copy(out_sh.at[pl.ds(sid * RPT, RPT)],
                    out_hbm.at[cid, pl.ds(sid * RPT, RPT)])


@functools.partial(
    pl.kernel,
    out_type=jax.ShapeDtypeStruct((NC, N_PAD, C), jnp.float32),
    mesh=_mesh,
    compiler_params=pltpu.CompilerParams(use_tc_tiling_on_sc=False, needs_layout_passes=False),
    scratch_types=[
        pltpu.VMEM((BE,), jnp.int32),
        pltpu.VMEM((BE,), jnp.int32),
        pltpu.VMEM((BE * H,), jnp.float32),
        pltpu.VMEM((BE, 2 * H), jnp.float32),
        pltpu.VMEM((BE, HC), jnp.float32),
        pltpu.VMEM((BE * H,), jnp.float32),
        pltpu.VMEM((BE, C), jnp.float32),
        pltpu.VMEM((L,), jnp.float32),
        pltpu.VMEM_SHARED((N_PAD, C), jnp.float32),
        pltpu.SemaphoreType.DMA,
        pltpu.SemaphoreType.DMA,
        pltpu.SemaphoreType.DMA,
    ],
)
def _e3(alpha_hbm, gmax_hbm, den_hbm, xl_hbm, src_hbm, dst_hbm, z32_hbm,
        out_hbm, srcv, dstv, albuf, denb, xlv, wbuf, ctb, gmv, out_sh,
        sem1, sem2, sem3):
    _e3_body(alpha_hbm, gmax_hbm, den_hbm, xl_hbm, src_hbm, dst_hbm, z32_hbm,
             out_hbm, srcv, dstv, albuf, denb, xlv, wbuf, ctb, gmv, out_sh,
             sem1, sem2, sem3)


def kernel(patch_embs, edge_index, edge_attr,
           Wl1, bl1, Wr1, br1, att1, bias1, g1, be1,
           Wl2, bl2, Wr2, br2, att2, bias2, g2, be2,
           Wl3, bl3, Wr3, br3, att3, bias3, g3, be3):
    del edge_attr
    ei = edge_index.astype(jnp.int32)
    pad = E_PAD - E
    padi = jnp.arange(pad, dtype=jnp.int32)
    src_pad = jnp.concatenate([ei[0], padi % N])
    dst_pad = jnp.concatenate([ei[1], N + (padi % 128)])
    z16 = jnp.zeros((N_PAD, 2 * H), jnp.float32)
    z32 = jnp.zeros((N_PAD, C), jnp.float32)

    params = [
        (Wl1, bl1, Wr1, br1, att1, bias1, g1, be1),
        (Wl2, bl2, Wr2, br2, att2, bias2, g2, be2),
        (Wl3, bl3, Wr3, br3, att3, bias3, g3, be3),
    ]
    x = patch_embs
    for layer, (wl, bl, wr, br, att, bias, g, be) in enumerate(params):
        xl, xr, aloop, loopmax = _t1(x, wl, bl, wr, br, att)
        alpha, wmax = _e1(xl, xr, src_pad, dst_pad, att)
        gmax = jnp.maximum(jnp.max(wmax, axis=(0, 2)), loopmax[0])
        gmax16 = jnp.tile(gmax, 2)
        asum = _e2(alpha, dst_pad, gmax16, z16)
        den16, lc = _t2(asum[0, :N, :H], asum[1, :N, :H], aloop,
                        gmax.reshape(1, H), xl)
        den_pad = jnp.concatenate(
            [den16, jnp.zeros((N_PAD - N, 2 * H), jnp.float32)], axis=0)
        outp = _e3(alpha, gmax16, den_pad, xl, src_pad, dst_pad, z32)
        y, st = _t3a(outp[0, :N], outp[1, :N], lc, bias, x, act=(layer < 2))
        x = _t3b(y, st, g, be)
    return x


# speedup vs baseline: 32.5335x; 1.5802x over previous
import functools

import jax
import jax.numpy as jnp
from jax import lax
from jax.experimental import pallas as pl
from jax.experimental.pallas import tpu as pltpu
from jax.experimental.pallas import tpu_sc as plsc

N = 50000
E = 800000
D = 32
H = 8
C = 32
HC = H * C

NC, NS, L = 2, 16, 16
NW = NC * NS
EPW = 25600
E_PAD = NW * EPW
BE = 64
NCHUNK = EPW // BE
N_PAD = 50176
RPT = N_PAD // NS

_mesh = plsc.VectorSubcoreMesh(core_axis_name="c", subcore_axis_name="s")


def _t1_body(x_ref, wl_ref, bl_ref, wr_ref, br_ref, att_ref,
             xl_ref, xr_ref, al_ref, lmax_ref):
    i = pl.program_id(0)
    x = x_ref[...]
    xl = jnp.dot(x, wl_ref[...], preferred_element_type=jnp.float32) + bl_ref[...]
    xr = jnp.dot(x, wr_ref[...], preferred_element_type=jnp.float32) + br_ref[...]
    xl_ref[...] = xl
    xr_ref[...] = xr
    r = x.shape[0]
    s = (xl + xr).reshape(r, H, C)
    m = jnp.where(s >= 0, s, 0.2 * s)
    al = jnp.sum(m * att_ref[...][None], axis=-1)
    al_ref[...] = al
    bm = jnp.max(al, axis=0, keepdims=True)
    prev = jnp.where(i == 0, jnp.full((1, H), -3.0e38, jnp.float32), lmax_ref[...])
    lmax_ref[...] = jnp.maximum(prev, bm)


def _t1(x, wl, bl, wr, br, att):
    r = 2000
    grid = N // r
    return pl.pallas_call(
        _t1_body,
        grid=(grid,),
        in_specs=[
            pl.BlockSpec((r, D), lambda i: (i, 0)),
            pl.BlockSpec((D, HC), lambda i: (0, 0)),
            pl.BlockSpec((1, HC), lambda i: (0, 0)),
            pl.BlockSpec((D, HC), lambda i: (0, 0)),
            pl.BlockSpec((1, HC), lambda i: (0, 0)),
            pl.BlockSpec((H, C), lambda i: (0, 0)),
        ],
        out_specs=[
            pl.BlockSpec((r, HC), lambda i: (i, 0)),
            pl.BlockSpec((r, HC), lambda i: (i, 0)),
            pl.BlockSpec((r, H), lambda i: (i, 0)),
            pl.BlockSpec((1, H), lambda i: (0, 0)),
        ],
        out_shape=[
            jax.ShapeDtypeStruct((N, HC), jnp.float32),
            jax.ShapeDtypeStruct((N, HC), jnp.float32),
            jax.ShapeDtypeStruct((N, H), jnp.float32),
            jax.ShapeDtypeStruct((1, H), jnp.float32),
        ],
    )(x, wl, bl.reshape(1, HC), wr, br.reshape(1, HC), att)


def _t2_body(a0_ref, a1_ref, al_ref, gmax_ref, xl_ref, den_ref, lc_ref):
    asum = a0_ref[...] + a1_ref[...]
    sae = jnp.exp(al_ref[...] - gmax_ref[...])
    den = 1.0 / (asum + sae)
    den_ref[...] = jnp.concatenate([den, jnp.zeros_like(den)], axis=1)
    w = sae * den
    r = w.shape[0]
    xl3 = xl_ref[...].reshape(r, H, C)
    lc_ref[...] = jnp.sum(w[:, :, None] * xl3, axis=1)


def _t2(a0, a1, aloop, gmax, xl):
    r = 400
    grid = N // r
    return pl.pallas_call(
        _t2_body,
        grid=(grid,),
        in_specs=[
            pl.BlockSpec((r, H), lambda i: (i, 0)),
            pl.BlockSpec((r, H), lambda i: (i, 0)),
            pl.BlockSpec((r, H), lambda i: (i, 0)),
            pl.BlockSpec((1, H), lambda i: (0, 0)),
            pl.BlockSpec((r, HC), lambda i: (i, 0)),
        ],
        out_specs=[
            pl.BlockSpec((r, 2 * H), lambda i: (i, 0)),
            pl.BlockSpec((r, C), lambda i: (i, 0)),
        ],
        out_shape=[
            jax.ShapeDtypeStruct((N, 2 * H), jnp.float32),
            jax.ShapeDtypeStruct((N, C), jnp.float32),
        ],
    )(a0, a1, aloop, gmax, xl)


def _t3a_body(p0_ref, p1_ref, lc_ref, bias_ref, xp_ref, y_ref, st_ref, *, act):
    i = pl.program_id(0)
    tot = (p0_ref[...] + p1_ref[...] + lc_ref[...]) * (1.0 / H) + bias_ref[...]
    if act:
        tot = jnp.where(tot >= 0, tot, 0.01 * tot)
    y = tot + xp_ref[...]
    y_ref[...] = y
    s = jnp.concatenate([jnp.sum(y, axis=0, keepdims=True),
                         jnp.sum(y * y, axis=0, keepdims=True)], axis=0)
    prev = jnp.where(i == 0, jnp.zeros((2, C), jnp.float32), st_ref[...])
    st_ref[...] = prev + s


def _t3a(p0, p1, lc, bias, xp, act):
    r = 2000
    grid = N // r
    return pl.pallas_call(
        functools.partial(_t3a_body, act=act),
        grid=(grid,),
        in_specs=[
            pl.BlockSpec((r, C), lambda i: (i, 0)),
            pl.BlockSpec((r, C), lambda i: (i, 0)),
            pl.BlockSpec((r, C), lambda i: (i, 0)),
            pl.BlockSpec((1, C), lambda i: (0, 0)),
            pl.BlockSpec((r, C), lambda i: (i, 0)),
        ],
        out_specs=[
            pl.BlockSpec((r, C), lambda i: (i, 0)),
            pl.BlockSpec((2, C), lambda i: (0, 0)),
        ],
        out_shape=[
            jax.ShapeDtypeStruct((N, C), jnp.float32),
            jax.ShapeDtypeStruct((2, C), jnp.float32),
        ],
    )(p0, p1, lc, bias.reshape(1, C), xp)


def _t3b_body(y_ref, st_ref, g_ref, be_ref, o_ref):
    st = st_ref[...]
    mu = st[0:1] / N
    var = st[1:2] / N - mu * mu
    scale = g_ref[...] / jnp.sqrt(var + 1e-5)
    o_ref[...] = (y_ref[...] - mu) * scale + be_ref[...]


def _t3b(y, st, g, be):
    r = 2000
    grid = N // r
    return pl.pallas_call(
        _t3b_body,
        grid=(grid,),
        in_specs=[
            pl.BlockSpec((r, C), lambda i: (i, 0)),
            pl.BlockSpec((2, C), lambda i: (0, 0)),
            pl.BlockSpec((1, C), lambda i: (0, 0)),
            pl.BlockSpec((1, C), lambda i: (0, 0)),
        ],
        out_specs=pl.BlockSpec((r, C), lambda i: (i, 0)),
        out_shape=jax.ShapeDtypeStruct((N, C), jnp.float32),
    )(y, st, g.reshape(1, C), be.reshape(1, C))


NPAIR = NCHUNK // 2


def _e1g_body(xl_hbm, xr_hbm, src_hbm, dstg_hbm, s_hbm,
              srcv0, dstv0, srcv1, dstv1, xlv0, xrv0, xlv1, xrv1,
              sv0, sv1, sl0, sr0, so0, sl1, sr1, so1):
    cid = lax.axis_index("c")
    sid = lax.axis_index("s")
    wid = sid * NC + cid
    base = wid * EPW
    bufs = ((srcv0, dstv0, xlv0, xrv0, sv0, sl0, sr0, so0),
            (srcv1, dstv1, xlv1, xrv1, sv1, sl1, sr1, so1))

    def start(gc, b):
        srcv, dstv, xlv, xrv, sv, sl, sr, so = bufs[b]
        off = base + gc * BE
        pltpu.sync_copy(src_hbm.at[pl.ds(off, BE)], srcv)
        pltpu.sync_copy(dstg_hbm.at[pl.ds(off, BE)], dstv)
        pltpu.async_copy(xl_hbm.at[srcv], xlv, sl)
        pltpu.async_copy(xr_hbm.at[dstv], xrv, sr)

    def wait_g(b):
        srcv, dstv, xlv, xrv, sv, sl, sr, so = bufs[b]
        pltpu.make_async_copy(xl_hbm.at[srcv], xlv, sl).wait()
        pltpu.make_async_copy(xr_hbm.at[dstv], xrv, sr).wait()

    def wait_o(b):
        srcv, dstv, xlv, xrv, sv, sl, sr, so = bufs[b]
        pltpu.make_async_copy(sv, s_hbm.at[pl.ds(base, BE)], so).wait()

    def addout(gc, b):
        srcv, dstv, xlv, xrv, sv, sl, sr, so = bufs[b]
        off = base + gc * BE

        def addrow(e, _):
            for k in range(HC // L):
                sv[e, pl.ds(k * L, L)] = (xlv[e, pl.ds(k * L, L)] +
                                          xrv[e, pl.ds(k * L, L)])
            return 0

        lax.fori_loop(0, BE, addrow, 0)
        pltpu.async_copy(sv, s_hbm.at[pl.ds(off, BE)], so)

    start(0, 0)
    start(1, 1)
    wait_g(0)
    addout(0, 0)
    start(2, 0)
    wait_g(1)
    addout(1, 1)
    start(3, 1)

    def body(j, _):
        a = 2 * j
        wait_g(0)
        wait_o(0)
        addout(a, 0)
        start(jnp.minimum(a + 2, NCHUNK - 1), 0)
        wait_g(1)
        wait_o(1)
        addout(a + 1, 1)
        start(jnp.minimum(a + 3, NCHUNK - 1), 1)
        return 0

    lax.fori_loop(1, NPAIR, body, 0)
    wait_g(0)
    wait_o(0)
    wait_g(1)
    wait_o(1)


@functools.partial(
    pl.kernel,
    out_type=jax.ShapeDtypeStruct((E_PAD, HC), jnp.float32),
    mesh=_mesh,
    compiler_params=pltpu.CompilerParams(use_tc_tiling_on_sc=False, needs_layout_passes=False),
    scratch_types=[
        pltpu.VMEM((BE,), jnp.int32),
        pltpu.VMEM((BE,), jnp.int32),
        pltpu.VMEM((BE,), jnp.int32),
        pltpu.VMEM((BE,), jnp.int32),
        pltpu.VMEM((BE, HC), jnp.float32),
        pltpu.VMEM((BE, HC), jnp.float32),
        pltpu.VMEM((BE, HC), jnp.float32),
        pltpu.VMEM((BE, HC), jnp.float32),
        pltpu.VMEM((BE, HC), jnp.float32),
        pltpu.VMEM((BE, HC), jnp.float32),
        pltpu.SemaphoreType.DMA,
        pltpu.SemaphoreType.DMA,
        pltpu.SemaphoreType.DMA,
        pltpu.SemaphoreType.DMA,
        pltpu.SemaphoreType.DMA,
        pltpu.SemaphoreType.DMA,
    ],
)
def _e1g(xl_hbm, xr_hbm, src_hbm, dstg_hbm, s_hbm,
         srcv0, dstv0, srcv1, dstv1, xlv0, xrv0, xlv1, xrv1,
         sv0, sv1, sl0, sr0, so0, sl1, sr1, so1):
    _e1g_body(xl_hbm, xr_hbm, src_hbm, dstg_hbm, s_hbm,
              srcv0, dstv0, srcv1, dstv1, xlv0, xrv0, xlv1, xrv1,
              sv0, sv1, sl0, sr0, so0, sl1, sr1, so1)


def _t1b_body(s_ref, att_ref, al_ref, mx_ref):
    i = pl.program_id(0)
    s = s_ref[...]
    m = jnp.where(s >= 0, s, 0.2 * s)
    r = s.shape[0]
    al = jnp.sum(m.reshape(r, H, C) * att_ref[...][None], axis=-1)
    al_ref[...] = al
    bm = jnp.max(al, axis=0, keepdims=True)
    prev = jnp.where(i == 0, jnp.full((1, H), -3.0e38, jnp.float32), mx_ref[...])
    mx_ref[...] = jnp.maximum(prev, bm)


def _t1b(s, att):
    r = 4096
    grid = E_PAD // r
    return pl.pallas_call(
        _t1b_body,
        grid=(grid,),
        in_specs=[
            pl.BlockSpec((r, HC), lambda i: (i, 0)),
            pl.BlockSpec((H, C), lambda i: (0, 0)),
        ],
        out_specs=[
            pl.BlockSpec((r, H), lambda i: (i, 0)),
            pl.BlockSpec((1, H), lambda i: (0, 0)),
        ],
        out_shape=[
            jax.ShapeDtypeStruct((E_PAD, H), jnp.float32),
            jax.ShapeDtypeStruct((1, H), jnp.float32),
        ],
    )(s, att)


def _e2_body(alpha_hbm, dst_hbm, gmax_hbm, z16_hbm, asum_hbm,
             dstv, albuf, upbuf, gmv, asum_sh, sem):
    cid = lax.axis_index("c")
    sid = lax.axis_index("s")
    wid = sid * NC + cid
    base = wid * EPW
    pltpu.sync_copy(z16_hbm.at[pl.ds(sid * RPT, RPT)],
                    asum_sh.at[pl.ds(sid * RPT, RPT)])
    plsc.subcore_barrier()
    pltpu.sync_copy(gmax_hbm, gmv)
    lanes = lax.iota(jnp.int32, L)
    rowpat = lanes >> 3
    colpat = lanes & 7
    zv = jnp.zeros((L,), jnp.float32)

    def zb(i, _):
        upbuf[i, :] = zv
        return 0

    lax.fori_loop(0, BE, zb, 0)
    gm = gmv[...]

    def chunk(gc, _):
        off = base + gc * BE
        pltpu.sync_copy(dst_hbm.at[pl.ds(off, BE)], dstv)
        pltpu.sync_copy(alpha_hbm.at[pl.ds(off * H, BE * H)], albuf)
        for j in range(BE * H // L):
            av = albuf[pl.ds(j * L, L)]
            ae = jnp.exp(av - gm)
            plsc.store_scatter(upbuf, [2 * j + rowpat, colpat], ae)
        pltpu.async_copy(upbuf, asum_sh.at[dstv], sem, add=True).wait()
        return 0

    lax.fori_loop(0, NCHUNK, chunk, 0)
    plsc.subcore_barrier()
    pltpu.sync_copy(asum_sh.at[pl.ds(sid * RPT, RPT)],
                    asum_hbm.at[cid, pl.ds(sid * RPT, RPT)])


@functools.partial(
    pl.kernel,
    out_type=jax.ShapeDtypeStruct((NC, N_PAD, 2 * H), jnp.float32),
    mesh=_mesh,
    compiler_params=pltpu.CompilerParams(use_tc_tiling_on_sc=False, needs_layout_passes=False),
    scratch_types=[
        pltpu.VMEM((BE,), jnp.int32),
        pltpu.VMEM((BE * H,), jnp.float32),
        pltpu.VMEM((BE, 2 * H), jnp.float32),
        pltpu.VMEM((L,), jnp.float32),
        pltpu.VMEM_SHARED((N_PAD, 2 * H), jnp.float32),
        pltpu.SemaphoreType.DMA,
    ],
)
def _e2(alpha_hbm, dst_hbm, gmax_hbm, z16_hbm, asum_hbm,
        dstv, albuf, upbuf, gmv, asum_sh, sem):
    _e2_body(alpha_hbm, dst_hbm, gmax_hbm, z16_hbm, asum_hbm,
             dstv, albuf, upbuf, gmv, asum_sh, sem)


def _e3_body(alpha_hbm, gmax_hbm, den_hbm, xl_hbm, src_hbm, dst_hbm, z32_hbm,
             out_hbm,
             srcv, dstv, albuf, denb, xlv, wbuf, ctb, gmv, out_sh,
             sem1, sem2, sem3):
    cid = lax.axis_index("c")
    sid = lax.axis_index("s")
    wid = sid * NC + cid
    base = wid * EPW
    pltpu.sync_copy(z32_hbm.at[pl.ds(sid * RPT, RPT)],
                    out_sh.at[pl.ds(sid * RPT, RPT)])
    plsc.subcore_barrier()
    pltpu.sync_copy(gmax_hbm, gmv)
    lanes = lax.iota(jnp.int32, L)
    rowpat = lanes >> 3
    colpat = lanes & 7
    gm = gmv[...]

    def chunk(gc, _):
        off = base + gc * BE
        pltpu.sync_copy(src_hbm.at[pl.ds(off, BE)], srcv)
        pltpu.sync_copy(dst_hbm.at[pl.ds(off, BE)], dstv)
        cp1 = pltpu.async_copy(xl_hbm.at[srcv], xlv, sem1)
        cp2 = pltpu.async_copy(den_hbm.at[dstv], denb, sem2)
        pltpu.sync_copy(alpha_hbm.at[pl.ds(off * H, BE * H)], albuf)
        cp2.wait()
        for j in range(BE * H // L):
            av = albuf[pl.ds(j * L, L)]
            ae = jnp.exp(av - gm)
            dv = plsc.load_gather(denb, [2 * j + rowpat, colpat])
            wbuf[pl.ds(j * L, L)] = ae * dv
        cp1.wait()

        def edge(ep, _):
            e0 = 2 * ep
            wv = wbuf[pl.ds(e0 * H, L)]
            for k in range(2):
                e = e0 + k
                c0 = jnp.zeros((L,), jnp.float32)
                c1 = jnp.zeros((L,), jnp.float32)
                for h in range(H):
                    ws = wv[k * H + h]
                    c0 = c0 + ws * xlv[e, pl.ds(h * C, L)]
                    c1 = c1 + ws * xlv[e, pl.ds(h * C + L, L)]
                ctb[e, pl.ds(0, L)] = c0
                ctb[e, pl.ds(L, L)] = c1
            return 0

        lax.fori_loop(0, BE // 2, edge, 0)
        pltpu.async_copy(ctb, out_sh.at[dstv], sem3, add=True).wait()
        return 0

    lax.fori_loop(0, NCHUNK, chunk, 0)
    plsc.subcore_barrier()
    pltpu.sync_copy(out_sh.at[pl.ds(sid * RPT, RPT)],
                    out_hbm.at[cid, pl.ds(sid * RPT, RPT)])


@functools.partial(
    pl.kernel,
    out_type=jax.ShapeDtypeStruct((NC, N_PAD, C), jnp.float32),
    mesh=_mesh,
    compiler_params=pltpu.CompilerParams(use_tc_tiling_on_sc=False, needs_layout_passes=False),
    scratch_types=[
        pltpu.VMEM((BE,), jnp.int32),
        pltpu.VMEM((BE,), jnp.int32),
        pltpu.VMEM((BE * H,), jnp.float32),
        pltpu.VMEM((BE, 2 * H), jnp.float32),
        pltpu.VMEM((BE, HC), jnp.float32),
        pltpu.VMEM((BE * H,), jnp.float32),
        pltpu.VMEM((BE, C), jnp.float32),
        pltpu.VMEM((L,), jnp.float32),
        pltpu.VMEM_SHARED((N_PAD, C), jnp.float32),
        pltpu.SemaphoreType.DMA,
        pltpu.SemaphoreType.DMA,
        pltpu.SemaphoreType.DMA,
    ],
)
def _e3(alpha_hbm, gmax_hbm, den_hbm, xl_hbm, src_hbm, dst_hbm, z32_hbm,
        out_hbm, srcv, dstv, albuf, denb, xlv, wbuf, ctb, gmv, out_sh,
        sem1, sem2, sem3):
    _e3_body(alpha_hbm, gmax_hbm, den_hbm, xl_hbm, src_hbm, dst_hbm, z32_hbm,
             out_hbm, srcv, dstv, albuf, denb, xlv, wbuf, ctb, gmv, out_sh,
             sem1, sem2, sem3)


def kernel(patch_embs, edge_index, edge_attr,
           Wl1, bl1, Wr1, br1, att1, bias1, g1, be1,
           Wl2, bl2, Wr2, br2, att2, bias2, g2, be2,
           Wl3, bl3, Wr3, br3, att3, bias3, g3, be3):
    del edge_attr
    ei = edge_index.astype(jnp.int32)
    pad = E_PAD - E
    padi = jnp.arange(pad, dtype=jnp.int32)
    src_pad = jnp.concatenate([ei[0], padi % N])
    dstg_pad = jnp.concatenate([ei[1], padi % N])
    dst_pad = jnp.concatenate([ei[1], N + (padi % 128)])
    z16 = jnp.zeros((N_PAD, 2 * H), jnp.float32)
    z32 = jnp.zeros((N_PAD, C), jnp.float32)

    params = [
        (Wl1, bl1, Wr1, br1, att1, bias1, g1, be1),
        (Wl2, bl2, Wr2, br2, att2, bias2, g2, be2),
        (Wl3, bl3, Wr3, br3, att3, bias3, g3, be3),
    ]
    x = patch_embs
    for layer, (wl, bl, wr, br, att, bias, g, be) in enumerate(params):
        xl, xr, aloop, loopmax = _t1(x, wl, bl, wr, br, att)
        s = _e1g(xl, xr, src_pad, dstg_pad)
        alpha2d, emax = _t1b(s, att)
        alpha = alpha2d.reshape(E_PAD * H)
        gmax = jnp.maximum(emax[0], loopmax[0])
        gmax16 = jnp.tile(gmax, 2)
        asum = _e2(alpha, dst_pad, gmax16, z16)
        den16, lc = _t2(asum[0, :N, :H], asum[1, :N, :H], aloop,
                        gmax.reshape(1, H), xl)
        den_pad = jnp.concatenate(
            [den16, jnp.zeros((N_PAD - N, 2 * H), jnp.float32)], axis=0)
        outp = _e3(alpha, gmax16, den_pad, xl, src_pad, dst_pad, z32)
        y, st = _t3a(outp[0, :N], outp[1, :N], lc, bias, x, act=(layer < 2))
        x = _t3b(y, st, g, be)
    return x


# speedup vs baseline: 41.2824x; 1.2689x over previous
import functools

import jax
import jax.numpy as jnp
from jax import lax
from jax.experimental import pallas as pl
from jax.experimental.pallas import tpu as pltpu
from jax.experimental.pallas import tpu_sc as plsc

N = 50000
E = 800000
D = 32
H = 8
C = 32
HC = H * C

NC, NS, L = 2, 16, 16
NW = NC * NS
EPW = 25600
E_PAD = NW * EPW
BE = 64
NCHUNK = EPW // BE
N_PAD = 50176
RPT = N_PAD // NS

_mesh = plsc.VectorSubcoreMesh(core_axis_name="c", subcore_axis_name="s")


def _t1_body(x_ref, wl_ref, bl_ref, wr_ref, br_ref, amat_ref,
             xl_ref, xr_ref, al_ref, lmax_ref):
    i = pl.program_id(0)
    x = x_ref[...]
    xl = jnp.dot(x, wl_ref[...], preferred_element_type=jnp.float32) + bl_ref[...]
    xr = jnp.dot(x, wr_ref[...], preferred_element_type=jnp.float32) + br_ref[...]
    xl_ref[...] = xl
    xr_ref[...] = xr
    s = xl + xr
    m = jnp.where(s >= 0, s, 0.2 * s)
    al = jnp.dot(m, amat_ref[...], preferred_element_type=jnp.float32)
    al_ref[...] = al
    bm = jnp.max(al, axis=0, keepdims=True)
    prev = jnp.where(i == 0, jnp.full((1, H), -3.0e38, jnp.float32), lmax_ref[...])
    lmax_ref[...] = jnp.maximum(prev, bm)


def _t1(x, wl, bl, wr, br, amat):
    r = 2000
    grid = N // r
    return pl.pallas_call(
        _t1_body,
        grid=(grid,),
        in_specs=[
            pl.BlockSpec((r, D), lambda i: (i, 0)),
            pl.BlockSpec((D, HC), lambda i: (0, 0)),
            pl.BlockSpec((1, HC), lambda i: (0, 0)),
            pl.BlockSpec((D, HC), lambda i: (0, 0)),
            pl.BlockSpec((1, HC), lambda i: (0, 0)),
            pl.BlockSpec((HC, H), lambda i: (0, 0)),
        ],
        out_specs=[
            pl.BlockSpec((r, HC), lambda i: (i, 0)),
            pl.BlockSpec((r, HC), lambda i: (i, 0)),
            pl.BlockSpec((r, H), lambda i: (i, 0)),
            pl.BlockSpec((1, H), lambda i: (0, 0)),
        ],
        out_shape=[
            jax.ShapeDtypeStruct((N, HC), jnp.float32),
            jax.ShapeDtypeStruct((N, HC), jnp.float32),
            jax.ShapeDtypeStruct((N, H), jnp.float32),
            jax.ShapeDtypeStruct((1, H), jnp.float32),
        ],
    )(x, wl, bl.reshape(1, HC), wr, br.reshape(1, HC), amat)


def _t2_body(a0_ref, a1_ref, al_ref, gmax_ref, xl_ref, den_ref, lc_ref):
    asum = a0_ref[...] + a1_ref[...]
    sae = jnp.exp(al_ref[...] - gmax_ref[...])
    den = 1.0 / (asum + sae)
    den_ref[...] = jnp.concatenate([den, jnp.zeros_like(den)], axis=1)
    w = sae * den
    xl2 = xl_ref[...]
    acc = w[:, 0:1] * xl2[:, 0:C]
    for h in range(1, H):
        acc = acc + w[:, h:h + 1] * xl2[:, h * C:(h + 1) * C]
    lc_ref[...] = acc


def _t2(a0, a1, aloop, gmax, xl):
    r = 400
    grid = N // r
    return pl.pallas_call(
        _t2_body,
        grid=(grid,),
        in_specs=[
            pl.BlockSpec((r, H), lambda i: (i, 0)),
            pl.BlockSpec((r, H), lambda i: (i, 0)),
            pl.BlockSpec((r, H), lambda i: (i, 0)),
            pl.BlockSpec((1, H), lambda i: (0, 0)),
            pl.BlockSpec((r, HC), lambda i: (i, 0)),
        ],
        out_specs=[
            pl.BlockSpec((r, 2 * H), lambda i: (i, 0)),
            pl.BlockSpec((r, C), lambda i: (i, 0)),
        ],
        out_shape=[
            jax.ShapeDtypeStruct((N, 2 * H), jnp.float32),
            jax.ShapeDtypeStruct((N, C), jnp.float32),
        ],
    )(a0, a1, aloop, gmax, xl)


def _t3a_body(p0_ref, p1_ref, lc_ref, bias_ref, xp_ref, y_ref, st_ref, *, act):
    i = pl.program_id(0)
    tot = (p0_ref[...] + p1_ref[...] + lc_ref[...]) * (1.0 / H) + bias_ref[...]
    if act:
        tot = jnp.where(tot >= 0, tot, 0.01 * tot)
    y = tot + xp_ref[...]
    y_ref[...] = y
    s = jnp.concatenate([jnp.sum(y, axis=0, keepdims=True),
                         jnp.sum(y * y, axis=0, keepdims=True)], axis=0)
    prev = jnp.where(i == 0, jnp.zeros((2, C), jnp.float32), st_ref[...])
    st_ref[...] = prev + s


def _t3a(p0, p1, lc, bias, xp, act):
    r = 2000
    grid = N // r
    return pl.pallas_call(
        functools.partial(_t3a_body, act=act),
        grid=(grid,),
        in_specs=[
            pl.BlockSpec((r, C), lambda i: (i, 0)),
            pl.BlockSpec((r, C), lambda i: (i, 0)),
            pl.BlockSpec((r, C), lambda i: (i, 0)),
            pl.BlockSpec((1, C), lambda i: (0, 0)),
            pl.BlockSpec((r, C), lambda i: (i, 0)),
        ],
        out_specs=[
            pl.BlockSpec((r, C), lambda i: (i, 0)),
            pl.BlockSpec((2, C), lambda i: (0, 0)),
        ],
        out_shape=[
            jax.ShapeDtypeStruct((N, C), jnp.float32),
            jax.ShapeDtypeStruct((2, C), jnp.float32),
        ],
    )(p0, p1, lc, bias.reshape(1, C), xp)


def _t3b_body(y_ref, st_ref, g_ref, be_ref, o_ref):
    st = st_ref[...]
    mu = st[0:1] / N
    var = st[1:2] / N - mu * mu
    scale = g_ref[...] / jnp.sqrt(var + 1e-5)
    o_ref[...] = (y_ref[...] - mu) * scale + be_ref[...]


def _t3b(y, st, g, be):
    r = 2000
    grid = N // r
    return pl.pallas_call(
        _t3b_body,
        grid=(grid,),
        in_specs=[
            pl.BlockSpec((r, C), lambda i: (i, 0)),
            pl.BlockSpec((2, C), lambda i: (0, 0)),
            pl.BlockSpec((1, C), lambda i: (0, 0)),
            pl.BlockSpec((1, C), lambda i: (0, 0)),
        ],
        out_specs=pl.BlockSpec((r, C), lambda i: (i, 0)),
        out_shape=jax.ShapeDtypeStruct((N, C), jnp.float32),
    )(y, st, g.reshape(1, C), be.reshape(1, C))


NPAIR = NCHUNK // 2


def _e1g_body(xl_hbm, xr_hbm, src_hbm, dstg_hbm, s_hbm,
              srcv0, dstv0, srcv1, dstv1, xlv0, xrv0, xlv1, xrv1,
              sv0, sv1, sl0, sr0, so0, sl1, sr1, so1):
    cid = lax.axis_index("c")
    sid = lax.axis_index("s")
    wid = sid * NC + cid
    base = wid * EPW
    bufs = ((srcv0, dstv0, xlv0, xrv0, sv0, sl0, sr0, so0),
            (srcv1, dstv1, xlv1, xrv1, sv1, sl1, sr1, so1))

    def start(gc, b):
        srcv, dstv, xlv, xrv, sv, sl, sr, so = bufs[b]
        off = base + gc * BE
        pltpu.sync_copy(src_hbm.at[pl.ds(off, BE)], srcv)
        pltpu.sync_copy(dstg_hbm.at[pl.ds(off, BE)], dstv)
        pltpu.async_copy(xl_hbm.at[srcv], xlv, sl)
        pltpu.async_copy(xr_hbm.at[dstv], xrv, sr)

    def wait_g(b):
        srcv, dstv, xlv, xrv, sv, sl, sr, so = bufs[b]
        pltpu.make_async_copy(xl_hbm.at[srcv], xlv, sl).wait()
        pltpu.make_async_copy(xr_hbm.at[dstv], xrv, sr).wait()

    def wait_o(b):
        srcv, dstv, xlv, xrv, sv, sl, sr, so = bufs[b]
        pltpu.make_async_copy(sv, s_hbm.at[pl.ds(base, BE)], so).wait()

    def addout(gc, b):
        srcv, dstv, xlv, xrv, sv, sl, sr, so = bufs[b]
        off = base + gc * BE

        def addrow(e, _):
            for k in range(HC // L):
                sv[e, pl.ds(k * L, L)] = (xlv[e, pl.ds(k * L, L)] +
                                          xrv[e, pl.ds(k * L, L)])
            return 0

        lax.fori_loop(0, BE, addrow, 0)
        pltpu.async_copy(sv, s_hbm.at[pl.ds(off, BE)], so)

    start(0, 0)
    start(1, 1)
    wait_g(0)
    addout(0, 0)
    start(2, 0)
    wait_g(1)
    addout(1, 1)
    start(3, 1)

    def body(j, _):
        a = 2 * j
        wait_g(0)
        wait_o(0)
        addout(a, 0)
        start(jnp.minimum(a + 2, NCHUNK - 1), 0)
        wait_g(1)
        wait_o(1)
        addout(a + 1, 1)
        start(jnp.minimum(a + 3, NCHUNK - 1), 1)
        return 0

    lax.fori_loop(1, NPAIR, body, 0)
    wait_g(0)
    wait_o(0)
    wait_g(1)
    wait_o(1)


@functools.partial(
    pl.kernel,
    out_type=jax.ShapeDtypeStruct((E_PAD, HC), jnp.float32),
    mesh=_mesh,
    compiler_params=pltpu.CompilerParams(use_tc_tiling_on_sc=False, needs_layout_passes=False),
    scratch_types=[
        pltpu.VMEM((BE,), jnp.int32),
        pltpu.VMEM((BE,), jnp.int32),
        pltpu.VMEM((BE,), jnp.int32),
        pltpu.VMEM((BE,), jnp.int32),
        pltpu.VMEM((BE, HC), jnp.float32),
        pltpu.VMEM((BE, HC), jnp.float32),
        pltpu.VMEM((BE, HC), jnp.float32),
        pltpu.VMEM((BE, HC), jnp.float32),
        pltpu.VMEM((BE, HC), jnp.float32),
        pltpu.VMEM((BE, HC), jnp.float32),
        pltpu.SemaphoreType.DMA,
        pltpu.SemaphoreType.DMA,
        pltpu.SemaphoreType.DMA,
        pltpu.SemaphoreType.DMA,
        pltpu.SemaphoreType.DMA,
        pltpu.SemaphoreType.DMA,
    ],
)
def _e1g(xl_hbm, xr_hbm, src_hbm, dstg_hbm, s_hbm,
         srcv0, dstv0, srcv1, dstv1, xlv0, xrv0, xlv1, xrv1,
         sv0, sv1, sl0, sr0, so0, sl1, sr1, so1):
    _e1g_body(xl_hbm, xr_hbm, src_hbm, dstg_hbm, s_hbm,
              srcv0, dstv0, srcv1, dstv1, xlv0, xrv0, xlv1, xrv1,
              sv0, sv1, sl0, sr0, so0, sl1, sr1, so1)


def _t1b_body(s_ref, amat_ref, al_ref, mx_ref):
    i = pl.program_id(0)
    s = s_ref[...]
    m = jnp.where(s >= 0, s, 0.2 * s)
    al = jnp.dot(m, amat_ref[...], preferred_element_type=jnp.float32)
    al_ref[...] = al
    bm = jnp.max(al, axis=0, keepdims=True)
    prev = jnp.where(i == 0, jnp.full((1, H), -3.0e38, jnp.float32), mx_ref[...])
    mx_ref[...] = jnp.maximum(prev, bm)


def _t1b(s, amat):
    r = 4096
    grid = E_PAD // r
    return pl.pallas_call(
        _t1b_body,
        grid=(grid,),
        in_specs=[
            pl.BlockSpec((r, HC), lambda i: (i, 0)),
            pl.BlockSpec((HC, H), lambda i: (0, 0)),
        ],
        out_specs=[
            pl.BlockSpec((r, H), lambda i: (i, 0)),
            pl.BlockSpec((1, H), lambda i: (0, 0)),
        ],
        out_shape=[
            jax.ShapeDtypeStruct((E_PAD, H), jnp.float32),
            jax.ShapeDtypeStruct((1, H), jnp.float32),
        ],
    )(s, amat)


def _e2_body(alpha_hbm, dst_hbm, gmax_hbm, z16_hbm, asum_hbm,
             dstv, albuf, upbuf, gmv, asum_sh, sem):
    cid = lax.axis_index("c")
    sid = lax.axis_index("s")
    wid = sid * NC + cid
    base = wid * EPW
    pltpu.sync_copy(z16_hbm.at[pl.ds(sid * RPT, RPT)],
                    asum_sh.at[pl.ds(sid * RPT, RPT)])
    plsc.subcore_barrier()
    pltpu.sync_copy(gmax_hbm, gmv)
    lanes = lax.iota(jnp.int32, L)
    rowpat = lanes >> 3
    colpat = lanes & 7
    zv = jnp.zeros((L,), jnp.float32)

    def zb(i, _):
        upbuf[i, :] = zv
        return 0

    lax.fori_loop(0, BE, zb, 0)
    gm = gmv[...]

    def chunk(gc, _):
        off = base + gc * BE
        pltpu.sync_copy(dst_hbm.at[pl.ds(off, BE)], dstv)
        pltpu.sync_copy(alpha_hbm.at[pl.ds(off * H, BE * H)], albuf)
        for j in range(BE * H // L):
            av = albuf[pl.ds(j * L, L)]
            ae = jnp.exp(av - gm)
            plsc.store_scatter(upbuf, [2 * j + rowpat, colpat], ae)
        pltpu.async_copy(upbuf, asum_sh.at[dstv], sem, add=True).wait()
        return 0

    lax.fori_loop(0, NCHUNK, chunk, 0)
    plsc.subcore_barrier()
    pltpu.sync_copy(asum_sh.at[pl.ds(sid * RPT, RPT)],
                    asum_hbm.at[cid, pl.ds(sid * RPT, RPT)])


@functools.partial(
    pl.kernel,
    out_type=jax.ShapeDtypeStruct((NC, N_PAD, 2 * H), jnp.float32),
    mesh=_mesh,
    compiler_params=pltpu.CompilerParams(use_tc_tiling_on_sc=False, needs_layout_passes=False),
    scratch_types=[
        pltpu.VMEM((BE,), jnp.int32),
        pltpu.VMEM((BE * H,), jnp.float32),
        pltpu.VMEM((BE, 2 * H), jnp.float32),
        pltpu.VMEM((L,), jnp.float32),
        pltpu.VMEM_SHARED((N_PAD, 2 * H), jnp.float32),
        pltpu.SemaphoreType.DMA,
    ],
)
def _e2(alpha_hbm, dst_hbm, gmax_hbm, z16_hbm, asum_hbm,
        dstv, albuf, upbuf, gmv, asum_sh, sem):
    _e2_body(alpha_hbm, dst_hbm, gmax_hbm, z16_hbm, asum_hbm,
             dstv, albuf, upbuf, gmv, asum_sh, sem)


def _e3_body(alpha_hbm, gmax_hbm, den_hbm, xl_hbm, src_hbm, dst_hbm, z32_hbm,
             out_hbm,
             srcv, dstv, albuf, denb, xlv, wbuf, ctb, gmv, out_sh,
             sem1, sem2, sem3):
    cid = lax.axis_index("c")
    sid = lax.axis_index("s")
    wid = sid * NC + cid
    base = wid * EPW
    pltpu.sync_copy(z32_hbm.at[pl.ds(sid * RPT, RPT)],
                    out_sh.at[pl.ds(sid * RPT, RPT)])
    plsc.subcore_barrier()
    pltpu.sync_copy(gmax_hbm, gmv)
    lanes = lax.iota(jnp.int32, L)
    rowpat = lanes >> 3
    colpat = lanes & 7
    gm = gmv[...]

    def chunk(gc, _):
        off = base + gc * BE
        pltpu.sync_copy(src_hbm.at[pl.ds(off, BE)], srcv)
        pltpu.sync_copy(dst_hbm.at[pl.ds(off, BE)], dstv)
        cp1 = pltpu.async_copy(xl_hbm.at[srcv], xlv, sem1)
        cp2 = pltpu.async_copy(den_hbm.at[dstv], denb, sem2)
        pltpu.sync_copy(alpha_hbm.at[pl.ds(off * H, BE * H)], albuf)
        cp2.wait()
        for j in range(BE * H // L):
            av = albuf[pl.ds(j * L, L)]
            ae = jnp.exp(av - gm)
            dv = plsc.load_gather(denb, [2 * j + rowpat, colpat])
            wbuf[pl.ds(j * L, L)] = ae * dv
        cp1.wait()

        def edge(ep, _):
            e0 = 2 * ep
            wv = wbuf[pl.ds(e0 * H, L)]
            for k in range(2):
                e = e0 + k
                c0 = jnp.zeros((L,), jnp.float32)
                c1 = jnp.zeros((L,), jnp.float32)
                for h in range(H):
                    ws = wv[k * H + h]
                    c0 = c0 + ws * xlv[e, pl.ds(h * C, L)]
                    c1 = c1 + ws * xlv[e, pl.ds(h * C + L, L)]
                ctb[e, pl.ds(0, L)] = c0
                ctb[e, pl.ds(L, L)] = c1
            return 0

        lax.fori_loop(0, BE // 2, edge, 0)
        pltpu.async_copy(ctb, out_sh.at[dstv], sem3, add=True).wait()
        return 0

    lax.fori_loop(0, NCHUNK, chunk, 0)
    plsc.subcore_barrier()
    pltpu.sync_copy(out_sh.at[pl.ds(sid * RPT, RPT)],
                    out_hbm.at[cid, pl.ds(sid * RPT, RPT)])


@functools.partial(
    pl.kernel,
    out_type=jax.ShapeDtypeStruct((NC, N_PAD, C), jnp.float32),
    mesh=_mesh,
    compiler_params=pltpu.CompilerParams(use_tc_tiling_on_sc=False, needs_layout_passes=False),
    scratch_types=[
        pltpu.VMEM((BE,), jnp.int32),
        pltpu.VMEM((BE,), jnp.int32),
        pltpu.VMEM((BE * H,), jnp.float32),
        pltpu.VMEM((BE, 2 * H), jnp.float32),
        pltpu.VMEM((BE, HC), jnp.float32),
        pltpu.VMEM((BE * H,), jnp.float32),
        pltpu.VMEM((BE, C), jnp.float32),
        pltpu.VMEM((L,), jnp.float32),
        pltpu.VMEM_SHARED((N_PAD, C), jnp.float32),
        pltpu.SemaphoreType.DMA,
        pltpu.SemaphoreType.DMA,
        pltpu.SemaphoreType.DMA,
    ],
)
def _e3(alpha_hbm, gmax_hbm, den_hbm, xl_hbm, src_hbm, dst_hbm, z32_hbm,
        out_hbm, srcv, dstv, albuf, denb, xlv, wbuf, ctb, gmv, out_sh,
        sem1, sem2, sem3):
    _e3_body(alpha_hbm, gmax_hbm, den_hbm, xl_hbm, src_hbm, dst_hbm, z32_hbm,
             out_hbm, srcv, dstv, albuf, denb, xlv, wbuf, ctb, gmv, out_sh,
             sem1, sem2, sem3)


def kernel(patch_embs, edge_index, edge_attr,
           Wl1, bl1, Wr1, br1, att1, bias1, g1, be1,
           Wl2, bl2, Wr2, br2, att2, bias2, g2, be2,
           Wl3, bl3, Wr3, br3, att3, bias3, g3, be3):
    del edge_attr
    ei = edge_index.astype(jnp.int32)
    pad = E_PAD - E
    padi = jnp.arange(pad, dtype=jnp.int32)
    src_pad = jnp.concatenate([ei[0], padi % N])
    dstg_pad = jnp.concatenate([ei[1], padi % N])
    dst_pad = jnp.concatenate([ei[1], N + (padi % 128)])
    z16 = jnp.zeros((N_PAD, 2 * H), jnp.float32)
    z32 = jnp.zeros((N_PAD, C), jnp.float32)

    params = [
        (Wl1, bl1, Wr1, br1, att1, bias1, g1, be1),
        (Wl2, bl2, Wr2, br2, att2, bias2, g2, be2),
        (Wl3, bl3, Wr3, br3, att3, bias3, g3, be3),
    ]
    rows = jnp.arange(HC, dtype=jnp.int32)
    x = patch_embs
    for layer, (wl, bl, wr, br, att, bias, g, be) in enumerate(params):
        amat = jnp.zeros((HC, H), jnp.float32).at[rows, rows // C].set(
            att.reshape(HC).astype(jnp.float32))
        xl, xr, aloop, loopmax = _t1(x, wl, bl, wr, br, amat)
        s = _e1g(xl, xr, src_pad, dstg_pad)
        alpha2d, emax = _t1b(s, amat)
        alpha = alpha2d.reshape(E_PAD * H)
        gmax = jnp.maximum(emax[0], loopmax[0])
        gmax16 = jnp.tile(gmax, 2)
        asum = _e2(alpha, dst_pad, gmax16, z16)
        den16, lc = _t2(asum[0, :N, :H], asum[1, :N, :H], aloop,
                        gmax.reshape(1, H), xl)
        den_pad = jnp.concatenate(
            [den16, jnp.zeros((N_PAD - N, 2 * H), jnp.float32)], axis=0)
        outp = _e3(alpha, gmax16, den_pad, xl, src_pad, dst_pad, z32)
        y, st = _t3a(outp[0, :N], outp[1, :N], lc, bias, x, act=(layer < 2))
        x = _t3b(y, st, g, be)
    return x


# speedup vs baseline: 44.9324x; 1.0884x over previous
import functools

import jax
import jax.numpy as jnp
from jax import lax
from jax.experimental import pallas as pl
from jax.experimental.pallas import tpu as pltpu
from jax.experimental.pallas import tpu_sc as plsc

N = 50000
E = 800000
D = 32
H = 8
C = 32
HC = H * C

NC, NS, L = 2, 16, 16
NW = NC * NS
EPW = 25600
E_PAD = NW * EPW
BE = 64
NCHUNK = EPW // BE
N_PAD = 50176
RPT = N_PAD // NS

_mesh = plsc.VectorSubcoreMesh(core_axis_name="c", subcore_axis_name="s")


def _t1_body(x_ref, wl_ref, bl_ref, wr_ref, br_ref, amat_ref,
             xl_ref, xr_ref, al_ref, lmax_ref):
    i = pl.program_id(0)
    x = x_ref[...]
    xl = jnp.dot(x, wl_ref[...], preferred_element_type=jnp.float32) + bl_ref[...]
    xr = jnp.dot(x, wr_ref[...], preferred_element_type=jnp.float32) + br_ref[...]
    xl_ref[...] = xl
    xr_ref[...] = xr
    s = xl + xr
    m = jnp.where(s >= 0, s, 0.2 * s)
    al = jnp.dot(m, amat_ref[...], preferred_element_type=jnp.float32)
    al_ref[...] = al
    bm = jnp.max(al, axis=0, keepdims=True)
    prev = jnp.where(i == 0, jnp.full((1, H), -3.0e38, jnp.float32), lmax_ref[...])
    lmax_ref[...] = jnp.maximum(prev, bm)


def _t1(x, wl, bl, wr, br, amat):
    r = 2000
    grid = N // r
    return pl.pallas_call(
        _t1_body,
        grid=(grid,),
        in_specs=[
            pl.BlockSpec((r, D), lambda i: (i, 0)),
            pl.BlockSpec((D, HC), lambda i: (0, 0)),
            pl.BlockSpec((1, HC), lambda i: (0, 0)),
            pl.BlockSpec((D, HC), lambda i: (0, 0)),
            pl.BlockSpec((1, HC), lambda i: (0, 0)),
            pl.BlockSpec((HC, H), lambda i: (0, 0)),
        ],
        out_specs=[
            pl.BlockSpec((r, HC), lambda i: (i, 0)),
            pl.BlockSpec((r, HC), lambda i: (i, 0)),
            pl.BlockSpec((r, H), lambda i: (i, 0)),
            pl.BlockSpec((1, H), lambda i: (0, 0)),
        ],
        out_shape=[
            jax.ShapeDtypeStruct((N, HC), jnp.float32),
            jax.ShapeDtypeStruct((N, HC), jnp.float32),
            jax.ShapeDtypeStruct((N, H), jnp.float32),
            jax.ShapeDtypeStruct((1, H), jnp.float32),
        ],
    )(x, wl, bl.reshape(1, HC), wr, br.reshape(1, HC), amat)


def _t2_body(a0_ref, a1_ref, al_ref, gmax_ref, xl_ref, den_ref, lc_ref):
    asum = a0_ref[...] + a1_ref[...]
    sae = jnp.exp(al_ref[...] - gmax_ref[...])
    den = 1.0 / (asum + sae)
    den_ref[...] = jnp.concatenate([den, jnp.zeros_like(den)], axis=1)
    w = sae * den
    xl2 = xl_ref[...]
    acc = w[:, 0:1] * xl2[:, 0:C]
    for h in range(1, H):
        acc = acc + w[:, h:h + 1] * xl2[:, h * C:(h + 1) * C]
    lc_ref[...] = acc


def _t2(a0, a1, aloop, gmax, xl):
    r = 400
    grid = N // r
    return pl.pallas_call(
        _t2_body,
        grid=(grid,),
        in_specs=[
            pl.BlockSpec((r, H), lambda i: (i, 0)),
            pl.BlockSpec((r, H), lambda i: (i, 0)),
            pl.BlockSpec((r, H), lambda i: (i, 0)),
            pl.BlockSpec((1, H), lambda i: (0, 0)),
            pl.BlockSpec((r, HC), lambda i: (i, 0)),
        ],
        out_specs=[
            pl.BlockSpec((r, 2 * H), lambda i: (i, 0)),
            pl.BlockSpec((r, C), lambda i: (i, 0)),
        ],
        out_shape=[
            jax.ShapeDtypeStruct((N, 2 * H), jnp.float32),
            jax.ShapeDtypeStruct((N, C), jnp.float32),
        ],
    )(a0, a1, aloop, gmax, xl)


def _t3a_body(p0_ref, p1_ref, lc_ref, bias_ref, xp_ref, y_ref, st_ref, *, act):
    i = pl.program_id(0)
    tot = (p0_ref[...] + p1_ref[...] + lc_ref[...]) * (1.0 / H) + bias_ref[...]
    if act:
        tot = jnp.where(tot >= 0, tot, 0.01 * tot)
    y = tot + xp_ref[...]
    y_ref[...] = y
    s = jnp.concatenate([jnp.sum(y, axis=0, keepdims=True),
                         jnp.sum(y * y, axis=0, keepdims=True)], axis=0)
    prev = jnp.where(i == 0, jnp.zeros((2, C), jnp.float32), st_ref[...])
    st_ref[...] = prev + s


def _t3a(p0, p1, lc, bias, xp, act):
    r = 2000
    grid = N // r
    return pl.pallas_call(
        functools.partial(_t3a_body, act=act),
        grid=(grid,),
        in_specs=[
            pl.BlockSpec((r, C), lambda i: (i, 0)),
            pl.BlockSpec((r, C), lambda i: (i, 0)),
            pl.BlockSpec((r, C), lambda i: (i, 0)),
            pl.BlockSpec((1, C), lambda i: (0, 0)),
            pl.BlockSpec((r, C), lambda i: (i, 0)),
        ],
        out_specs=[
            pl.BlockSpec((r, C), lambda i: (i, 0)),
            pl.BlockSpec((2, C), lambda i: (0, 0)),
        ],
        out_shape=[
            jax.ShapeDtypeStruct((N, C), jnp.float32),
            jax.ShapeDtypeStruct((2, C), jnp.float32),
        ],
    )(p0, p1, lc, bias.reshape(1, C), xp)


def _t3b_body(y_ref, st_ref, g_ref, be_ref, o_ref):
    st = st_ref[...]
    mu = st[0:1] / N
    var = st[1:2] / N - mu * mu
    scale = g_ref[...] / jnp.sqrt(var + 1e-5)
    o_ref[...] = (y_ref[...] - mu) * scale + be_ref[...]


def _t3b(y, st, g, be):
    r = 2000
    grid = N // r
    return pl.pallas_call(
        _t3b_body,
        grid=(grid,),
        in_specs=[
            pl.BlockSpec((r, C), lambda i: (i, 0)),
            pl.BlockSpec((2, C), lambda i: (0, 0)),
            pl.BlockSpec((1, C), lambda i: (0, 0)),
            pl.BlockSpec((1, C), lambda i: (0, 0)),
        ],
        out_specs=pl.BlockSpec((r, C), lambda i: (i, 0)),
        out_shape=jax.ShapeDtypeStruct((N, C), jnp.float32),
    )(y, st, g.reshape(1, C), be.reshape(1, C))


NPAIR = NCHUNK // 2


def _e1g_body(xl_hbm, xr_hbm, src_hbm, dstg_hbm, s_hbm,
              srcv0, dstv0, srcv1, dstv1, xlv0, xrv0, xlv1, xrv1,
              sv0, sv1, sl0, sr0, so0, sl1, sr1, so1):
    cid = lax.axis_index("c")
    sid = lax.axis_index("s")
    wid = sid * NC + cid
    base = wid * EPW
    bufs = ((srcv0, dstv0, xlv0, xrv0, sv0, sl0, sr0, so0),
            (srcv1, dstv1, xlv1, xrv1, sv1, sl1, sr1, so1))

    def start(gc, b):
        srcv, dstv, xlv, xrv, sv, sl, sr, so = bufs[b]
        off = base + gc * BE
        pltpu.sync_copy(src_hbm.at[pl.ds(off, BE)], srcv)
        pltpu.sync_copy(dstg_hbm.at[pl.ds(off, BE)], dstv)
        pltpu.async_copy(xl_hbm.at[srcv], xlv, sl)
        pltpu.async_copy(xr_hbm.at[dstv], xrv, sr)

    def wait_g(b):
        srcv, dstv, xlv, xrv, sv, sl, sr, so = bufs[b]
        pltpu.make_async_copy(xl_hbm.at[srcv], xlv, sl).wait()
        pltpu.make_async_copy(xr_hbm.at[dstv], xrv, sr).wait()

    def wait_o(b):
        srcv, dstv, xlv, xrv, sv, sl, sr, so = bufs[b]
        pltpu.make_async_copy(sv, s_hbm.at[pl.ds(base, BE)], so).wait()

    def addout(gc, b):
        srcv, dstv, xlv, xrv, sv, sl, sr, so = bufs[b]
        off = base + gc * BE

        def addrow(e, _):
            for k in range(HC // L):
                sv[e, pl.ds(k * L, L)] = (xlv[e, pl.ds(k * L, L)] +
                                          xrv[e, pl.ds(k * L, L)])
            return 0

        lax.fori_loop(0, BE, addrow, 0)
        pltpu.async_copy(sv, s_hbm.at[pl.ds(off, BE)], so)

    start(0, 0)
    start(1, 1)
    wait_g(0)
    addout(0, 0)
    start(2, 0)
    wait_g(1)
    addout(1, 1)
    start(3, 1)

    def body(j, _):
        a = 2 * j
        wait_g(0)
        wait_o(0)
        addout(a, 0)
        start(jnp.minimum(a + 2, NCHUNK - 1), 0)
        wait_g(1)
        wait_o(1)
        addout(a + 1, 1)
        start(jnp.minimum(a + 3, NCHUNK - 1), 1)
        return 0

    lax.fori_loop(1, NPAIR, body, 0)
    wait_g(0)
    wait_o(0)
    wait_g(1)
    wait_o(1)


@functools.partial(
    pl.kernel,
    out_type=jax.ShapeDtypeStruct((E_PAD, HC), jnp.float32),
    mesh=_mesh,
    compiler_params=pltpu.CompilerParams(use_tc_tiling_on_sc=False, needs_layout_passes=False),
    scratch_types=[
        pltpu.VMEM((BE,), jnp.int32),
        pltpu.VMEM((BE,), jnp.int32),
        pltpu.VMEM((BE,), jnp.int32),
        pltpu.VMEM((BE,), jnp.int32),
        pltpu.VMEM((BE, HC), jnp.float32),
        pltpu.VMEM((BE, HC), jnp.float32),
        pltpu.VMEM((BE, HC), jnp.float32),
        pltpu.VMEM((BE, HC), jnp.float32),
        pltpu.VMEM((BE, HC), jnp.float32),
        pltpu.VMEM((BE, HC), jnp.float32),
        pltpu.SemaphoreType.DMA,
        pltpu.SemaphoreType.DMA,
        pltpu.SemaphoreType.DMA,
        pltpu.SemaphoreType.DMA,
        pltpu.SemaphoreType.DMA,
        pltpu.SemaphoreType.DMA,
    ],
)
def _e1g(xl_hbm, xr_hbm, src_hbm, dstg_hbm, s_hbm,
         srcv0, dstv0, srcv1, dstv1, xlv0, xrv0, xlv1, xrv1,
         sv0, sv1, sl0, sr0, so0, sl1, sr1, so1):
    _e1g_body(xl_hbm, xr_hbm, src_hbm, dstg_hbm, s_hbm,
              srcv0, dstv0, srcv1, dstv1, xlv0, xrv0, xlv1, xrv1,
              sv0, sv1, sl0, sr0, so0, sl1, sr1, so1)


def _t1b_body(s_ref, amat_ref, al_ref, mx_ref):
    i = pl.program_id(0)
    s = s_ref[...]
    m = jnp.where(s >= 0, s, 0.2 * s)
    al = jnp.dot(m, amat_ref[...], preferred_element_type=jnp.float32)
    al_ref[...] = al
    bm = jnp.max(al, axis=0, keepdims=True)
    prev = jnp.where(i == 0, jnp.full((1, H), -3.0e38, jnp.float32), mx_ref[...])
    mx_ref[...] = jnp.maximum(prev, bm)


def _t1b(s, amat):
    r = 4096
    grid = E_PAD // r
    return pl.pallas_call(
        _t1b_body,
        grid=(grid,),
        in_specs=[
            pl.BlockSpec((r, HC), lambda i: (i, 0)),
            pl.BlockSpec((HC, H), lambda i: (0, 0)),
        ],
        out_specs=[
            pl.BlockSpec((r, H), lambda i: (i, 0)),
            pl.BlockSpec((1, H), lambda i: (0, 0)),
        ],
        out_shape=[
            jax.ShapeDtypeStruct((E_PAD, H), jnp.float32),
            jax.ShapeDtypeStruct((1, H), jnp.float32),
        ],
    )(s, amat)


BE2 = 512
NCH2 = EPW // BE2


def _e2_body(alpha_hbm, dst_hbm, gmax_hbm, z16_hbm, asum_hbm,
             dstv0, albuf0, upbuf0, dstv1, albuf1, upbuf1,
             gmv, asum_sh, sem0, sem1):
    cid = lax.axis_index("c")
    sid = lax.axis_index("s")
    wid = sid * NC + cid
    base = wid * EPW
    pltpu.sync_copy(z16_hbm.at[pl.ds(sid * RPT, RPT)],
                    asum_sh.at[pl.ds(sid * RPT, RPT)])
    plsc.subcore_barrier()
    pltpu.sync_copy(gmax_hbm, gmv)
    lanes = lax.iota(jnp.int32, L)
    rowpat = lanes >> 3
    colpat = lanes & 7
    zv = jnp.zeros((L,), jnp.float32)
    bufs = ((dstv0, albuf0, upbuf0, sem0), (dstv1, albuf1, upbuf1, sem1))

    def zb(i, _):
        upbuf0[i, :] = zv
        upbuf1[i, :] = zv
        return 0

    lax.fori_loop(0, BE2, zb, 0)
    gm = gmv[...]

    def compute(gc, b):
        dstv, albuf, upbuf, sem = bufs[b]
        off = base + gc * BE2
        pltpu.sync_copy(dst_hbm.at[pl.ds(off, BE2)], dstv)
        pltpu.sync_copy(alpha_hbm.at[pl.ds(off * H, BE2 * H)], albuf)

        def grp(j, _):
            av = albuf[pl.ds(j * L, L)]
            ae = jnp.exp(av - gm)
            plsc.store_scatter(upbuf, [2 * j + rowpat, colpat], ae)
            return 0

        lax.fori_loop(0, BE2 * H // L, grp, 0)
        pltpu.async_copy(upbuf, asum_sh.at[dstv], sem, add=True)

    def wait_sc(b):
        dstv, albuf, upbuf, sem = bufs[b]
        pltpu.make_async_copy(upbuf, asum_sh.at[dstv], sem).wait()

    compute(0, 0)
    compute(1, 1)

    def chunk(j, _):
        a = 2 * j
        wait_sc(0)
        compute(a, 0)
        wait_sc(1)
        compute(a + 1, 1)
        return 0

    lax.fori_loop(1, NCH2 // 2, chunk, 0)
    wait_sc(0)
    wait_sc(1)
    plsc.subcore_barrier()
    pltpu.sync_copy(asum_sh.at[pl.ds(sid * RPT, RPT)],
                    asum_hbm.at[cid, pl.ds(sid * RPT, RPT)])


@functools.partial(
    pl.kernel,
    out_type=jax.ShapeDtypeStruct((NC, N_PAD, 2 * H), jnp.float32),
    mesh=_mesh,
    compiler_params=pltpu.CompilerParams(use_tc_tiling_on_sc=False, needs_layout_passes=False),
    scratch_types=[
        pltpu.VMEM((BE2,), jnp.int32),
        pltpu.VMEM((BE2 * H,), jnp.float32),
        pltpu.VMEM((BE2, 2 * H), jnp.float32),
        pltpu.VMEM((BE2,), jnp.int32),
        pltpu.VMEM((BE2 * H,), jnp.float32),
        pltpu.VMEM((BE2, 2 * H), jnp.float32),
        pltpu.VMEM((L,), jnp.float32),
        pltpu.VMEM_SHARED((N_PAD, 2 * H), jnp.float32),
        pltpu.SemaphoreType.DMA,
        pltpu.SemaphoreType.DMA,
    ],
)
def _e2(alpha_hbm, dst_hbm, gmax_hbm, z16_hbm, asum_hbm,
        dstv0, albuf0, upbuf0, dstv1, albuf1, upbuf1, gmv, asum_sh,
        sem0, sem1):
    _e2_body(alpha_hbm, dst_hbm, gmax_hbm, z16_hbm, asum_hbm,
             dstv0, albuf0, upbuf0, dstv1, albuf1, upbuf1,
             gmv, asum_sh, sem0, sem1)


def _e3_body(alpha_hbm, gmax_hbm, den_hbm, xl_hbm, src_hbm, dst_hbm, z32_hbm,
             out_hbm,
             srcv, dstv, albuf, denb, xlv, wbuf, dsts0, ctb0, dsts1, ctb1,
             gmv, out_sh, gl, gd, ga, sc0, sc1):
    cid = lax.axis_index("c")
    sid = lax.axis_index("s")
    wid = sid * NC + cid
    base = wid * EPW
    pltpu.sync_copy(z32_hbm.at[pl.ds(sid * RPT, RPT)],
                    out_sh.at[pl.ds(sid * RPT, RPT)])
    plsc.subcore_barrier()
    pltpu.sync_copy(gmax_hbm, gmv)
    lanes = lax.iota(jnp.int32, L)
    rowpat = lanes >> 3
    colpat = lanes & 7
    gm = gmv[...]
    sbufs = ((dsts0, ctb0, sc0), (dsts1, ctb1, sc1))

    def wait_sc(b):
        dsts, ctb, sc = sbufs[b]
        pltpu.make_async_copy(ctb, out_sh.at[dsts], sc).wait()

    def compute(gc, b, first):
        dsts, ctb, sc = sbufs[b]
        off = base + gc * BE
        pltpu.sync_copy(src_hbm.at[pl.ds(off, BE)], srcv)
        pltpu.sync_copy(dst_hbm.at[pl.ds(off, BE)], dstv)
        cp1 = pltpu.async_copy(xl_hbm.at[srcv], xlv, gl)
        cp2 = pltpu.async_copy(den_hbm.at[dstv], denb, gd)
        cp3 = pltpu.async_copy(alpha_hbm.at[pl.ds(off * H, BE * H)], albuf, ga)
        cp2.wait()
        cp3.wait()

        def wgrp(j, _):
            av = albuf[pl.ds(j * L, L)]
            ae = jnp.exp(av - gm)
            dv = plsc.load_gather(denb, [2 * j + rowpat, colpat])
            wbuf[pl.ds(j * L, L)] = ae * dv
            return 0

        lax.fori_loop(0, BE * H // L, wgrp, 0)
        cp1.wait()
        if not first:
            wait_sc(b)

        def edge(ep, _):
            e0 = 2 * ep
            wv = wbuf[pl.ds(e0 * H, L)]
            for k in range(2):
                e = e0 + k
                c0 = jnp.zeros((L,), jnp.float32)
                c1 = jnp.zeros((L,), jnp.float32)
                for h in range(H):
                    ws = wv[k * H + h]
                    c0 = c0 + ws * xlv[e, pl.ds(h * C, L)]
                    c1 = c1 + ws * xlv[e, pl.ds(h * C + L, L)]
                ctb[e, pl.ds(0, L)] = c0
                ctb[e, pl.ds(L, L)] = c1
            return 0

        lax.fori_loop(0, BE // 2, edge, 0)

        def icp(k, _):
            dsts[pl.ds(k * L, L)] = dstv[pl.ds(k * L, L)]
            return 0

        lax.fori_loop(0, BE // L, icp, 0)
        pltpu.async_copy(ctb, out_sh.at[dsts], sc, add=True)

    compute(0, 0, True)
    compute(1, 1, True)

    def chunk(j, _):
        a = 2 * j
        compute(a, 0, False)
        compute(a + 1, 1, False)
        return 0

    lax.fori_loop(1, NPAIR, chunk, 0)
    wait_sc(0)
    wait_sc(1)
    plsc.subcore_barrier()
    pltpu.sync_copy(out_sh.at[pl.ds(sid * RPT, RPT)],
                    out_hbm.at[cid, pl.ds(sid * RPT, RPT)])


@functools.partial(
    pl.kernel,
    out_type=jax.ShapeDtypeStruct((NC, N_PAD, C), jnp.float32),
    mesh=_mesh,
    compiler_params=pltpu.CompilerParams(use_tc_tiling_on_sc=False, needs_layout_passes=False),
    scratch_types=[
        pltpu.VMEM((BE,), jnp.int32),
        pltpu.VMEM((BE,), jnp.int32),
        pltpu.VMEM((BE * H,), jnp.float32),
        pltpu.VMEM((BE, 2 * H), jnp.float32),
        pltpu.VMEM((BE, HC), jnp.float32),
        pltpu.VMEM((BE * H,), jnp.float32),
        pltpu.VMEM((BE,), jnp.int32),
        pltpu.VMEM((BE, C), jnp.float32),
        pltpu.VMEM((BE,), jnp.int32),
        pltpu.VMEM((BE, C), jnp.float32),
        pltpu.VMEM((L,), jnp.float32),
        pltpu.VMEM_SHARED((N_PAD, C), jnp.float32),
        pltpu.SemaphoreType.DMA,
        pltpu.SemaphoreType.DMA,
        pltpu.SemaphoreType.DMA,
        pltpu.SemaphoreType.DMA,
        pltpu.SemaphoreType.DMA,
    ],
)
def _e3(alpha_hbm, gmax_hbm, den_hbm, xl_hbm, src_hbm, dst_hbm, z32_hbm,
        out_hbm,
        srcv, dstv, albuf, denb, xlv, wbuf, dsts0, ctb0, dsts1, ctb1,
        gmv, out_sh, gl, gd, ga, sc0, sc1):
    _e3_body(alpha_hbm, gmax_hbm, den_hbm, xl_hbm, src_hbm, dst_hbm, z32_hbm,
             out_hbm,
             srcv, dstv, albuf, denb, xlv, wbuf, dsts0, ctb0, dsts1, ctb1,
             gmv, out_sh, gl, gd, ga, sc0, sc1)


def kernel(patch_embs, edge_index, edge_attr,
           Wl1, bl1, Wr1, br1, att1, bias1, g1, be1,
           Wl2, bl2, Wr2, br2, att2, bias2, g2, be2,
           Wl3, bl3, Wr3, br3, att3, bias3, g3, be3):
    del edge_attr
    ei = edge_index.astype(jnp.int32)
    pad = E_PAD - E
    padi = jnp.arange(pad, dtype=jnp.int32)
    src_pad = jnp.concatenate([ei[0], padi % N])
    dstg_pad = jnp.concatenate([ei[1], padi % N])
    dst_pad = jnp.concatenate([ei[1], N + (padi % 128)])
    z16 = jnp.zeros((N_PAD, 2 * H), jnp.float32)
    z32 = jnp.zeros((N_PAD, C), jnp.float32)

    params = [
        (Wl1, bl1, Wr1, br1, att1, bias1, g1, be1),
        (Wl2, bl2, Wr2, br2, att2, bias2, g2, be2),
        (Wl3, bl3, Wr3, br3, att3, bias3, g3, be3),
    ]
    rows = jnp.arange(HC, dtype=jnp.int32)
    x = patch_embs
    for layer, (wl, bl, wr, br, att, bias, g, be) in enumerate(params):
        amat = jnp.zeros((HC, H), jnp.float32).at[rows, rows // C].set(
            att.reshape(HC).astype(jnp.float32))
        xl, xr, aloop, loopmax = _t1(x, wl, bl, wr, br, amat)
        s = _e1g(xl, xr, src_pad, dstg_pad)
        alpha2d, emax = _t1b(s, amat)
        alpha = alpha2d.reshape(E_PAD * H)
        gmax = jnp.maximum(emax[0], loopmax[0])
        gmax16 = jnp.tile(gmax, 2)
        asum = _e2(alpha, dst_pad, gmax16, z16)
        den16, lc = _t2(asum[0, :N, :H], asum[1, :N, :H], aloop,
                        gmax.reshape(1, H), xl)
        den_pad = jnp.concatenate(
            [den16, jnp.zeros((N_PAD - N, 2 * H), jnp.float32)], axis=0)
        outp = _e3(alpha, gmax16, den_pad, xl, src_pad, dst_pad, z32)
        y, st = _t3a(outp[0, :N], outp[1, :N], lc, bias, x, act=(layer < 2))
        x = _t3b(y, st, g, be)
    return x


# speedup vs baseline: 48.4880x; 1.0791x over previous
import functools

import jax
import jax.numpy as jnp
from jax import lax
from jax.experimental import pallas as pl
from jax.experimental.pallas import tpu as pltpu
from jax.experimental.pallas import tpu_sc as plsc

N = 50000
E = 800000
D = 32
H = 8
C = 32
HC = H * C

NC, NS, L = 2, 16, 16
NW = NC * NS
EPW = 25600
E_PAD = NW * EPW
BE = 64
NCHUNK = EPW // BE
N_PAD = 50176
RPT = N_PAD // NS

_mesh = plsc.VectorSubcoreMesh(core_axis_name="c", subcore_axis_name="s")


def _t1_body(x_ref, wl_ref, bl_ref, wr_ref, br_ref, amat_ref,
             xl_ref, xr_ref, al_ref, lmax_ref):
    i = pl.program_id(0)
    x = x_ref[...]
    xl = jnp.dot(x, wl_ref[...], preferred_element_type=jnp.float32) + bl_ref[...]
    xr = jnp.dot(x, wr_ref[...], preferred_element_type=jnp.float32) + br_ref[...]
    xl_ref[...] = xl
    xr_ref[...] = xr
    s = xl + xr
    m = jnp.where(s >= 0, s, 0.2 * s)
    al = jnp.dot(m, amat_ref[...], preferred_element_type=jnp.float32)
    al_ref[...] = al
    bm = jnp.max(al, axis=0, keepdims=True)
    prev = jnp.where(i == 0, jnp.full((1, H), -3.0e38, jnp.float32), lmax_ref[...])
    lmax_ref[...] = jnp.maximum(prev, bm)


def _t1(x, wl, bl, wr, br, amat):
    r = 2000
    grid = N // r
    return pl.pallas_call(
        _t1_body,
        grid=(grid,),
        in_specs=[
            pl.BlockSpec((r, D), lambda i: (i, 0)),
            pl.BlockSpec((D, HC), lambda i: (0, 0)),
            pl.BlockSpec((1, HC), lambda i: (0, 0)),
            pl.BlockSpec((D, HC), lambda i: (0, 0)),
            pl.BlockSpec((1, HC), lambda i: (0, 0)),
            pl.BlockSpec((HC, H), lambda i: (0, 0)),
        ],
        out_specs=[
            pl.BlockSpec((r, HC), lambda i: (i, 0)),
            pl.BlockSpec((r, HC), lambda i: (i, 0)),
            pl.BlockSpec((r, H), lambda i: (i, 0)),
            pl.BlockSpec((1, H), lambda i: (0, 0)),
        ],
        out_shape=[
            jax.ShapeDtypeStruct((N, HC), jnp.float32),
            jax.ShapeDtypeStruct((N, HC), jnp.float32),
            jax.ShapeDtypeStruct((N, H), jnp.float32),
            jax.ShapeDtypeStruct((1, H), jnp.float32),
        ],
    )(x, wl, bl.reshape(1, HC), wr, br.reshape(1, HC), amat)


def _t2_body(a0_ref, a1_ref, al_ref, gmax_ref, xl_ref, den_ref, lc_ref):
    asum = a0_ref[...] + a1_ref[...]
    sae = jnp.exp(al_ref[...] - gmax_ref[...])
    den = 1.0 / (asum + sae)
    den_ref[...] = jnp.concatenate([den, jnp.zeros_like(den)], axis=1)
    w = sae * den
    xl2 = xl_ref[...]
    acc = w[:, 0:1] * xl2[:, 0:C]
    for h in range(1, H):
        acc = acc + w[:, h:h + 1] * xl2[:, h * C:(h + 1) * C]
    lc_ref[...] = acc


def _t2(a0, a1, aloop, gmax, xl):
    r = 400
    grid = N // r
    return pl.pallas_call(
        _t2_body,
        grid=(grid,),
        in_specs=[
            pl.BlockSpec((r, H), lambda i: (i, 0)),
            pl.BlockSpec((r, H), lambda i: (i, 0)),
            pl.BlockSpec((r, H), lambda i: (i, 0)),
            pl.BlockSpec((1, H), lambda i: (0, 0)),
            pl.BlockSpec((r, HC), lambda i: (i, 0)),
        ],
        out_specs=[
            pl.BlockSpec((r, 2 * H), lambda i: (i, 0)),
            pl.BlockSpec((r, C), lambda i: (i, 0)),
        ],
        out_shape=[
            jax.ShapeDtypeStruct((N, 2 * H), jnp.float32),
            jax.ShapeDtypeStruct((N, C), jnp.float32),
        ],
    )(a0, a1, aloop, gmax, xl)


def _t3a_body(p0_ref, p1_ref, lc_ref, bias_ref, xp_ref, y_ref, st_ref, *, act):
    i = pl.program_id(0)
    tot = (p0_ref[...] + p1_ref[...] + lc_ref[...]) * (1.0 / H) + bias_ref[...]
    if act:
        tot = jnp.where(tot >= 0, tot, 0.01 * tot)
    y = tot + xp_ref[...]
    y_ref[...] = y
    s = jnp.concatenate([jnp.sum(y, axis=0, keepdims=True),
                         jnp.sum(y * y, axis=0, keepdims=True)], axis=0)
    prev = jnp.where(i == 0, jnp.zeros((2, C), jnp.float32), st_ref[...])
    st_ref[...] = prev + s


def _t3a(p0, p1, lc, bias, xp, act):
    r = 2000
    grid = N // r
    return pl.pallas_call(
        functools.partial(_t3a_body, act=act),
        grid=(grid,),
        in_specs=[
            pl.BlockSpec((r, C), lambda i: (i, 0)),
            pl.BlockSpec((r, C), lambda i: (i, 0)),
            pl.BlockSpec((r, C), lambda i: (i, 0)),
            pl.BlockSpec((1, C), lambda i: (0, 0)),
            pl.BlockSpec((r, C), lambda i: (i, 0)),
        ],
        out_specs=[
            pl.BlockSpec((r, C), lambda i: (i, 0)),
            pl.BlockSpec((2, C), lambda i: (0, 0)),
        ],
        out_shape=[
            jax.ShapeDtypeStruct((N, C), jnp.float32),
            jax.ShapeDtypeStruct((2, C), jnp.float32),
        ],
    )(p0, p1, lc, bias.reshape(1, C), xp)


def _t3b_body(y_ref, st_ref, g_ref, be_ref, o_ref):
    st = st_ref[...]
    mu = st[0:1] / N
    var = st[1:2] / N - mu * mu
    scale = g_ref[...] / jnp.sqrt(var + 1e-5)
    o_ref[...] = (y_ref[...] - mu) * scale + be_ref[...]


def _t3b(y, st, g, be):
    r = 2000
    grid = N // r
    return pl.pallas_call(
        _t3b_body,
        grid=(grid,),
        in_specs=[
            pl.BlockSpec((r, C), lambda i: (i, 0)),
            pl.BlockSpec((2, C), lambda i: (0, 0)),
            pl.BlockSpec((1, C), lambda i: (0, 0)),
            pl.BlockSpec((1, C), lambda i: (0, 0)),
        ],
        out_specs=pl.BlockSpec((r, C), lambda i: (i, 0)),
        out_shape=jax.ShapeDtypeStruct((N, C), jnp.float32),
    )(y, st, g.reshape(1, C), be.reshape(1, C))


NPAIR = NCHUNK // 2


def _e1g_body(xl_hbm, xr_hbm, src_hbm, dstg_hbm, s_hbm,
              srcv0, dstv0, srcv1, dstv1, xlv0, xrv0, xlv1, xrv1,
              sv0, sv1, sl0, sr0, so0, sl1, sr1, so1):
    cid = lax.axis_index("c")
    sid = lax.axis_index("s")
    wid = sid * NC + cid
    base = wid * EPW
    bufs = ((srcv0, dstv0, xlv0, xrv0, sv0, sl0, sr0, so0),
            (srcv1, dstv1, xlv1, xrv1, sv1, sl1, sr1, so1))

    def start(gc, b):
        srcv, dstv, xlv, xrv, sv, sl, sr, so = bufs[b]
        off = base + gc * BE
        pltpu.sync_copy(src_hbm.at[pl.ds(off, BE)], srcv)
        pltpu.sync_copy(dstg_hbm.at[pl.ds(off, BE)], dstv)
        pltpu.async_copy(xl_hbm.at[srcv], xlv, sl)
        pltpu.async_copy(xr_hbm.at[dstv], xrv, sr)

    def wait_g(b):
        srcv, dstv, xlv, xrv, sv, sl, sr, so = bufs[b]
        pltpu.make_async_copy(xl_hbm.at[srcv], xlv, sl).wait()
        pltpu.make_async_copy(xr_hbm.at[dstv], xrv, sr).wait()

    def wait_o(b):
        srcv, dstv, xlv, xrv, sv, sl, sr, so = bufs[b]
        pltpu.make_async_copy(sv, s_hbm.at[pl.ds(base, BE)], so).wait()

    def addout(gc, b):
        srcv, dstv, xlv, xrv, sv, sl, sr, so = bufs[b]
        off = base + gc * BE

        def addrow(e, _):
            for k in range(HC // L):
                sv[e, pl.ds(k * L, L)] = (xlv[e, pl.ds(k * L, L)] +
                                          xrv[e, pl.ds(k * L, L)])
            return 0

        lax.fori_loop(0, BE, addrow, 0)
        pltpu.async_copy(sv, s_hbm.at[pl.ds(off, BE)], so)

    start(0, 0)
    start(1, 1)
    wait_g(0)
    addout(0, 0)
    start(2, 0)
    wait_g(1)
    addout(1, 1)
    start(3, 1)

    def body(j, _):
        a = 2 * j
        wait_g(0)
        wait_o(0)
        addout(a, 0)
        start(jnp.minimum(a + 2, NCHUNK - 1), 0)
        wait_g(1)
        wait_o(1)
        addout(a + 1, 1)
        start(jnp.minimum(a + 3, NCHUNK - 1), 1)
        return 0

    lax.fori_loop(1, NPAIR, body, 0)
    wait_g(0)
    wait_o(0)
    wait_g(1)
    wait_o(1)


@functools.partial(
    pl.kernel,
    out_type=jax.ShapeDtypeStruct((E_PAD, HC), jnp.float32),
    mesh=_mesh,
    compiler_params=pltpu.CompilerParams(use_tc_tiling_on_sc=False, needs_layout_passes=False),
    scratch_types=[
        pltpu.VMEM((BE,), jnp.int32),
        pltpu.VMEM((BE,), jnp.int32),
        pltpu.VMEM((BE,), jnp.int32),
        pltpu.VMEM((BE,), jnp.int32),
        pltpu.VMEM((BE, HC), jnp.float32),
        pltpu.VMEM((BE, HC), jnp.float32),
        pltpu.VMEM((BE, HC), jnp.float32),
        pltpu.VMEM((BE, HC), jnp.float32),
        pltpu.VMEM((BE, HC), jnp.float32),
        pltpu.VMEM((BE, HC), jnp.float32),
        pltpu.SemaphoreType.DMA,
        pltpu.SemaphoreType.DMA,
        pltpu.SemaphoreType.DMA,
        pltpu.SemaphoreType.DMA,
        pltpu.SemaphoreType.DMA,
        pltpu.SemaphoreType.DMA,
    ],
)
def _e1g(xl_hbm, xr_hbm, src_hbm, dstg_hbm, s_hbm,
         srcv0, dstv0, srcv1, dstv1, xlv0, xrv0, xlv1, xrv1,
         sv0, sv1, sl0, sr0, so0, sl1, sr1, so1):
    _e1g_body(xl_hbm, xr_hbm, src_hbm, dstg_hbm, s_hbm,
              srcv0, dstv0, srcv1, dstv1, xlv0, xrv0, xlv1, xrv1,
              sv0, sv1, sl0, sr0, so0, sl1, sr1, so1)


def _t1b_body(s_ref, amat_ref, al_ref, mx_ref):
    i = pl.program_id(0)
    s = s_ref[...]
    m = jnp.where(s >= 0, s, 0.2 * s)
    al = lax.dot_general(amat_ref[...], m, (((0,), (1,)), ((), ())),
                         preferred_element_type=jnp.float32)
    al_ref[...] = al
    bm = jnp.max(al, axis=1, keepdims=True)
    prev = jnp.where(i == 0, jnp.full((H, 1), -3.0e38, jnp.float32), mx_ref[...])
    mx_ref[...] = jnp.maximum(prev, bm)


def _t1b(s, amat):
    r = 4096
    grid = E_PAD // r
    return pl.pallas_call(
        _t1b_body,
        grid=(grid,),
        in_specs=[
            pl.BlockSpec((r, HC), lambda i: (i, 0)),
            pl.BlockSpec((HC, H), lambda i: (0, 0)),
        ],
        out_specs=[
            pl.BlockSpec((H, r), lambda i: (0, i)),
            pl.BlockSpec((H, 1), lambda i: (0, 0)),
        ],
        out_shape=[
            jax.ShapeDtypeStruct((H, E_PAD), jnp.float32),
            jax.ShapeDtypeStruct((H, 1), jnp.float32),
        ],
    )(s, amat)


BE2 = 512
NCH2 = EPW // BE2


def _e2_body(alpha_hbm, dst_hbm, gmax_hbm, z16_hbm, asum_hbm,
             dstv0, albuf0, upbuf0, dstv1, albuf1, upbuf1,
             gmv, asum_sh, sem0, sem1):
    cid = lax.axis_index("c")
    sid = lax.axis_index("s")
    wid = sid * NC + cid
    base = wid * EPW
    pltpu.sync_copy(z16_hbm.at[pl.ds(sid * RPT, RPT)],
                    asum_sh.at[pl.ds(sid * RPT, RPT)])
    plsc.subcore_barrier()
    pltpu.sync_copy(gmax_hbm, gmv)
    lanes = lax.iota(jnp.int32, L)
    zv = jnp.zeros((L,), jnp.float32)
    bufs = ((dstv0, albuf0, upbuf0, sem0), (dstv1, albuf1, upbuf1, sem1))

    def zb(i, _):
        upbuf0[i, :] = zv
        upbuf1[i, :] = zv
        return 0

    lax.fori_loop(0, BE2, zb, 0)
    gm = gmv[...]

    def compute(gc, b):
        dstv, albuf, upbuf, sem = bufs[b]
        off = base + gc * BE2
        pltpu.sync_copy(dst_hbm.at[pl.ds(off, BE2)], dstv)
        pltpu.sync_copy(alpha_hbm.at[:, pl.ds(off, BE2)], albuf)

        for h in range(H):
            gmh = gm[h]
            colh = jnp.full((L,), h, jnp.int32)

            def grp(j, _, h=h, gmh=gmh, colh=colh):
                rows = j * L + lanes
                av = albuf[h, pl.ds(j * L, L)]
                ae = jnp.exp(av - gmh)
                plsc.store_scatter(upbuf, [rows, colh], ae)
                return 0

            lax.fori_loop(0, BE2 // L, grp, 0)
        pltpu.async_copy(upbuf, asum_sh.at[dstv], sem, add=True)

    def wait_sc(b):
        dstv, albuf, upbuf, sem = bufs[b]
        pltpu.make_async_copy(upbuf, asum_sh.at[dstv], sem).wait()

    compute(0, 0)
    compute(1, 1)

    def chunk(j, _):
        a = 2 * j
        wait_sc(0)
        compute(a, 0)
        wait_sc(1)
        compute(a + 1, 1)
        return 0

    lax.fori_loop(1, NCH2 // 2, chunk, 0)
    wait_sc(0)
    wait_sc(1)
    plsc.subcore_barrier()
    pltpu.sync_copy(asum_sh.at[pl.ds(sid * RPT, RPT)],
                    asum_hbm.at[cid, pl.ds(sid * RPT, RPT)])


@functools.partial(
    pl.kernel,
    out_type=jax.ShapeDtypeStruct((NC, N_PAD, 2 * H), jnp.float32),
    mesh=_mesh,
    compiler_params=pltpu.CompilerParams(use_tc_tiling_on_sc=False, needs_layout_passes=False),
    scratch_types=[
        pltpu.VMEM((BE2,), jnp.int32),
        pltpu.VMEM((H, BE2), jnp.float32),
        pltpu.VMEM((BE2, 2 * H), jnp.float32),
        pltpu.VMEM((BE2,), jnp.int32),
        pltpu.VMEM((H, BE2), jnp.float32),
        pltpu.VMEM((BE2, 2 * H), jnp.float32),
        pltpu.VMEM((L,), jnp.float32),
        pltpu.VMEM_SHARED((N_PAD, 2 * H), jnp.float32),
        pltpu.SemaphoreType.DMA,
        pltpu.SemaphoreType.DMA,
    ],
)
def _e2(alpha_hbm, dst_hbm, gmax_hbm, z16_hbm, asum_hbm,
        dstv0, albuf0, upbuf0, dstv1, albuf1, upbuf1, gmv, asum_sh,
        sem0, sem1):
    _e2_body(alpha_hbm, dst_hbm, gmax_hbm, z16_hbm, asum_hbm,
             dstv0, albuf0, upbuf0, dstv1, albuf1, upbuf1,
             gmv, asum_sh, sem0, sem1)


def _e3_body(alpha_hbm, gmax_hbm, den_hbm, xl_hbm, src_hbm, dst_hbm, z32_hbm,
             out_hbm,
             srcv, dstv, albuf, denb, xlv, wbuf, dsts0, ctb0, dsts1, ctb1,
             gmv, out_sh, gl, gd, ga, sc0, sc1):
    cid = lax.axis_index("c")
    sid = lax.axis_index("s")
    wid = sid * NC + cid
    base = wid * EPW
    pltpu.sync_copy(z32_hbm.at[pl.ds(sid * RPT, RPT)],
                    out_sh.at[pl.ds(sid * RPT, RPT)])
    plsc.subcore_barrier()
    pltpu.sync_copy(gmax_hbm, gmv)
    lanes = lax.iota(jnp.int32, L)
    gm = gmv[...]
    sbufs = ((dsts0, ctb0, sc0), (dsts1, ctb1, sc1))

    def wait_sc(b):
        dsts, ctb, sc = sbufs[b]
        pltpu.make_async_copy(ctb, out_sh.at[dsts], sc).wait()

    def compute(gc, b, first):
        dsts, ctb, sc = sbufs[b]
        off = base + gc * BE
        pltpu.sync_copy(src_hbm.at[pl.ds(off, BE)], srcv)
        pltpu.sync_copy(dst_hbm.at[pl.ds(off, BE)], dstv)
        cp1 = pltpu.async_copy(xl_hbm.at[srcv], xlv, gl)
        cp2 = pltpu.async_copy(den_hbm.at[dstv], denb, gd)
        cp3 = pltpu.async_copy(alpha_hbm.at[:, pl.ds(off, BE)], albuf, ga)
        cp2.wait()
        cp3.wait()

        for h in range(H):
            gmh = gm[h]
            colh = jnp.full((L,), h, jnp.int32)

            def wgrp(j, _, h=h, gmh=gmh, colh=colh):
                rows = j * L + lanes
                av = albuf[h, pl.ds(j * L, L)]
                ae = jnp.exp(av - gmh)
                dv = plsc.load_gather(denb, [rows, colh])
                plsc.store_scatter(wbuf, [rows * H + h], ae * dv)
                return 0

            lax.fori_loop(0, BE // L, wgrp, 0)
        cp1.wait()
        if not first:
            wait_sc(b)

        def edge(ep, _):
            e0 = 2 * ep
            wv = wbuf[pl.ds(e0 * H, L)]
            for k in range(2):
                e = e0 + k
                c0 = jnp.zeros((L,), jnp.float32)
                c1 = jnp.zeros((L,), jnp.float32)
                for h in range(H):
                    ws = wv[k * H + h]
                    c0 = c0 + ws * xlv[e, pl.ds(h * C, L)]
                    c1 = c1 + ws * xlv[e, pl.ds(h * C + L, L)]
                ctb[e, pl.ds(0, L)] = c0
                ctb[e, pl.ds(L, L)] = c1
            return 0

        lax.fori_loop(0, BE // 2, edge, 0)

        def icp(k, _):
            dsts[pl.ds(k * L, L)] = dstv[pl.ds(k * L, L)]
            return 0

        lax.fori_loop(0, BE // L, icp, 0)
        pltpu.async_copy(ctb, out_sh.at[dsts], sc, add=True)

    compute(0, 0, True)
    compute(1, 1, True)

    def chunk(j, _):
        a = 2 * j
        compute(a, 0, False)
        compute(a + 1, 1, False)
        return 0

    lax.fori_loop(1, NPAIR, chunk, 0)
    wait_sc(0)
    wait_sc(1)
    plsc.subcore_barrier()
    pltpu.sync_copy(out_sh.at[pl.ds(sid * RPT, RPT)],
                    out_hbm.at[cid, pl.ds(sid * RPT, RPT)])


@functools.partial(
    pl.kernel,
    out_type=jax.ShapeDtypeStruct((NC, N_PAD, C), jnp.float32),
    mesh=_mesh,
    compiler_params=pltpu.CompilerParams(use_tc_tiling_on_sc=False, needs_layout_passes=False),
    scratch_types=[
        pltpu.VMEM((BE,), jnp.int32),
        pltpu.VMEM((BE,), jnp.int32),
        pltpu.VMEM((H, BE), jnp.float32),
        pltpu.VMEM((BE, 2 * H), jnp.float32),
        pltpu.VMEM((BE, HC), jnp.float32),
        pltpu.VMEM((BE * H,), jnp.float32),
        pltpu.VMEM((BE,), jnp.int32),
        pltpu.VMEM((BE, C), jnp.float32),
        pltpu.VMEM((BE,), jnp.int32),
        pltpu.VMEM((BE, C), jnp.float32),
        pltpu.VMEM((L,), jnp.float32),
        pltpu.VMEM_SHARED((N_PAD, C), jnp.float32),
        pltpu.SemaphoreType.DMA,
        pltpu.SemaphoreType.DMA,
        pltpu.SemaphoreType.DMA,
        pltpu.SemaphoreType.DMA,
        pltpu.SemaphoreType.DMA,
    ],
)
def _e3(alpha_hbm, gmax_hbm, den_hbm, xl_hbm, src_hbm, dst_hbm, z32_hbm,
        out_hbm,
        srcv, dstv, albuf, denb, xlv, wbuf, dsts0, ctb0, dsts1, ctb1,
        gmv, out_sh, gl, gd, ga, sc0, sc1):
    _e3_body(alpha_hbm, gmax_hbm, den_hbm, xl_hbm, src_hbm, dst_hbm, z32_hbm,
             out_hbm,
             srcv, dstv, albuf, denb, xlv, wbuf, dsts0, ctb0, dsts1, ctb1,
             gmv, out_sh, gl, gd, ga, sc0, sc1)


def kernel(patch_embs, edge_index, edge_attr,
           Wl1, bl1, Wr1, br1, att1, bias1, g1, be1,
           Wl2, bl2, Wr2, br2, att2, bias2, g2, be2,
           Wl3, bl3, Wr3, br3, att3, bias3, g3, be3):
    del edge_attr
    ei = edge_index.astype(jnp.int32)
    pad = E_PAD - E
    padi = jnp.arange(pad, dtype=jnp.int32)
    src_pad = jnp.concatenate([ei[0], padi % N])
    dstg_pad = jnp.concatenate([ei[1], padi % N])
    dst_pad = jnp.concatenate([ei[1], N + (padi % 128)])
    z16 = jnp.zeros((N_PAD, 2 * H), jnp.float32)
    z32 = jnp.zeros((N_PAD, C), jnp.float32)

    params = [
        (Wl1, bl1, Wr1, br1, att1, bias1, g1, be1),
        (Wl2, bl2, Wr2, br2, att2, bias2, g2, be2),
        (Wl3, bl3, Wr3, br3, att3, bias3, g3, be3),
    ]
    rows = jnp.arange(HC, dtype=jnp.int32)
    x = patch_embs
    for layer, (wl, bl, wr, br, att, bias, g, be) in enumerate(params):
        amat = jnp.zeros((HC, H), jnp.float32).at[rows, rows // C].set(
            att.reshape(HC).astype(jnp.float32))
        xl, xr, aloop, loopmax = _t1(x, wl, bl, wr, br, amat)
        s = _e1g(xl, xr, src_pad, dstg_pad)
        alpha, emax = _t1b(s, amat)
        gmax = jnp.maximum(emax[:, 0], loopmax[0])
        gmax16 = jnp.tile(gmax, 2)
        asum = _e2(alpha, dst_pad, gmax16, z16)
        den16, lc = _t2(asum[0, :N, :H], asum[1, :N, :H], aloop,
                        gmax.reshape(1, H), xl)
        den_pad = jnp.concatenate(
            [den16, jnp.zeros((N_PAD - N, 2 * H), jnp.float32)], axis=0)
        outp = _e3(alpha, gmax16, den_pad, xl, src_pad, dst_pad, z32)
        y, st = _t3a(outp[0, :N], outp[1, :N], lc, bias, x, act=(layer < 2))
        x = _t3b(y, st, g, be)
    return x
